# baseline (speedup 1.0000x reference)
import functools

import jax
import jax.numpy as jnp
import numpy as np
from jax import lax
from jax.experimental import pallas as pl
from jax.experimental.pallas import tpu as pltpu
from jax.experimental.pallas import tpu_sc as plsc

f32 = jnp.float32
i32 = jnp.int32

MIN_NORM = 1e-15
C = float(np.float32(1e-6))
SQRT_C = float(np.sqrt(np.float32(1e-6)))
MAXNORM = float(np.float32(1.0 - 0.004) / np.float32(SQRT_C))

N_NODE = 10000
N_PAD = 10240
A_PAD = 10112
R_PAD = 512
IN_DIM = 128
NC, NS = 2, 16
NW = NC * NS
CHUNK = 128

_SC_PARAMS = pltpu.CompilerParams(needs_layout_passes=False)


def _expmap0_scale(ns2):
    ns = jnp.sqrt(ns2)
    un = jnp.maximum(ns, MIN_NORM)
    arg = jnp.clip(SQRT_C * un, -15.0, 15.0)
    g = jnp.tanh(arg) / (SQRT_C * un)
    gn = jnp.maximum(g * ns, MIN_NORM)
    return g * jnp.where(gn > MAXNORM, MAXNORM / gn, 1.0)


def _logmap0_scale(yn2):
    yn = jnp.maximum(jnp.sqrt(yn2), MIN_NORM)
    z = jnp.clip(SQRT_C * yn, -1 + 1e-05, 1 - 1e-05)
    art = 0.5 * jnp.log((1.0 + z) / (1.0 - z))
    return art / (yn * SQRT_C)


def _prep_body(h_ref, ws_ref, rec_ref):
    h = h_ref[...]
    a = jnp.dot(h, ws_ref[...], preferred_element_type=f32)
    ns2 = jnp.sum(h * h, axis=-1, keepdims=True)
    fs = _expmap0_scale(ns2)
    lane = lax.broadcasted_iota(i32, (1, 16), 1)
    rec = a + jnp.where(lane == 8, fs, 0.0) + jnp.where(lane == 9, ns2, 0.0)
    rec_ref[...] = lax.slice(rec, (0, 0), (rec.shape[0], 10))


def _prep_call(h_pad, ws_pad):
    nblk = N_PAD // 512
    return pl.pallas_call(
        _prep_body,
        grid=(nblk,),
        in_specs=[
            pl.BlockSpec((512, IN_DIM), lambda i: (i, 0)),
            pl.BlockSpec((IN_DIM, 16), lambda i: (0, 0)),
        ],
        out_specs=pl.BlockSpec((512, 10), lambda i: (i, 0)),
        out_shape=jax.ShapeDtypeStruct((N_PAD, 10), f32),
    )(h_pad, ws_pad)


def _relprep_body(rp_ref, wr_ref, wqr_ref, b_ref, q_ref, rrec_ref, qrec_ref):
    rp = rp_ref[...]
    a = jnp.dot(rp, wr_ref[...], preferred_element_type=f32)
    nr2 = jnp.sum(rp * rp, axis=-1, keepdims=True)
    gr = _expmap0_scale(nr2)
    lane = lax.broadcasted_iota(i32, (1, 16), 1)
    rec = a + jnp.where(lane == 8, gr, 0.0) + jnp.where(lane == 9, nr2, 0.0)
    rrec_ref[...] = lax.slice(rec, (0, 0), (512, 10))
    col = lax.broadcasted_iota(i32, (512, R_PAD), 1)
    oh = jnp.where(col == q_ref[...], 1.0, 0.0).astype(f32)
    hq = jnp.dot(oh, rp, preferred_element_type=f32)
    qrec = jnp.dot(hq, wqr_ref[...], preferred_element_type=f32) + b_ref[...]
    qrec_ref[...] = lax.slice(qrec, (0, 0), (512, 8))


def _relprep_call(rela_pad, wr_pad, wqr_pad, b_pad, q_rel32):
    return pl.pallas_call(
        _relprep_body,
        in_specs=[
            pl.BlockSpec((R_PAD, IN_DIM), lambda: (0, 0)),
            pl.BlockSpec((IN_DIM, 16), lambda: (0, 0)),
            pl.BlockSpec((IN_DIM, 16), lambda: (0, 0)),
            pl.BlockSpec((1, 16), lambda: (0, 0)),
            pl.BlockSpec((512, 1), lambda: (0, 0)),
        ],
        out_specs=[
            pl.BlockSpec((R_PAD, 10), lambda: (0, 0)),
            pl.BlockSpec((512, 8), lambda: (0, 0)),
        ],
        out_shape=[
            jax.ShapeDtypeStruct((R_PAD, 10), f32),
            jax.ShapeDtypeStruct((512, 8), f32),
        ],
    )(rela_pad, wr_pad, wqr_pad, b_pad, q_rel32)


def _make_gather(e_pad):
    per_tile = e_pad // NW
    nchunk = per_tile // CHUNK
    nrow = e_pad // CHUNK
    mesh = plsc.VectorSubcoreMesh(core_axis_name="c", subcore_axis_name="s")

    nerow = per_tile // 64

    @functools.partial(
        pl.kernel,
        out_type=jax.ShapeDtypeStruct((16 * nrow, CHUNK), f32),
        mesh=mesh,
        compiler_params=_SC_PARAMS,
        scratch_types=[
            pltpu.VMEM((N_PAD * 10,), f32),
            pltpu.VMEM((R_PAD * 10,), f32),
            pltpu.VMEM((512 * 8,), f32),
            pltpu.VMEM((16,), i32),
            pltpu.VMEM((16, CHUNK), f32),
            pltpu.VMEM((16, CHUNK), f32),
            pltpu.SemaphoreType.DMA,
        ],
    )
    def gather(nflat, rflat, qflat, e3, o2,
               nrec_t, rrec_t, qrec_t, rg_v, ebuf, fbuf, sem):
        sid = lax.axis_index("s").astype(i32)
        cid = lax.axis_index("c").astype(i32)
        wid = sid * np.int32(NC) + cid
        rgbase = wid * np.int32(nerow)
        rbase = wid * np.int32(nchunk)

        pltpu.sync_copy(nflat, nrec_t)
        pltpu.sync_copy(rflat, rrec_t)
        pltpu.sync_copy(qflat, qrec_t)

        zv = jnp.zeros((16,), f32)
        for r in range(12, 16):
            for g in range(CHUNK // 16):
                fbuf[r, pl.ds(g * 16, 16)] = zv

        ii = lax.iota(i32, 16)

        @pl.loop(np.int32(0), np.int32(nchunk // 8), step=np.int32(1))
        def _super(sg):
            rg_v[pl.ds(0, 16)] = ii + (rgbase + sg.astype(i32) * np.int32(16))
            pltpu.async_copy(e3.at[rg_v], ebuf, sem).wait()

            @pl.loop(np.int32(0), np.int32(8), step=np.int32(1))
            def _chunk(tj):
                t = tj.astype(i32)
                for g in range(CHUNK // 16):
                    sl = pl.ds(g * 16, 16)
                    row16 = lax.broadcast(
                        t * np.int32(2) + np.int32(g // 4), (16,))
                    lane16 = ii * np.int32(2) + np.int32((g % 4) * 32)
                    p = plsc.bitcast(
                        plsc.load_gather(ebuf, [row16, lane16]), i32)
                    ridx16 = plsc.bitcast(
                        plsc.load_gather(ebuf, [row16, lane16 + np.int32(1)]), i32)
                    sub = lax.shift_right_logical(p, 9)
                    rel = lax.bitwise_and(p, np.int32(511))
                    s10 = sub * np.int32(10)
                    r10 = rel * np.int32(10)
                    q8 = ridx16 * np.int32(8)
                    for k in range(8):
                        v = (plsc.load_gather(nrec_t, [s10 + np.int32(k)])
                             + plsc.load_gather(rrec_t, [r10 + np.int32(k)])
                             + plsc.load_gather(qrec_t, [q8 + np.int32(k)]))
                        fbuf[k, sl] = v
                    fbuf[8, sl] = plsc.load_gather(nrec_t, [s10 + np.int32(8)])
                    fbuf[9, sl] = plsc.load_gather(nrec_t, [s10 + np.int32(9)])
                    fbuf[10, sl] = plsc.load_gather(rrec_t, [r10 + np.int32(8)])
                    fbuf[11, sl] = plsc.load_gather(rrec_t, [r10 + np.int32(9)])
                oidx = ii * np.int32(nrow) + (rbase + sg.astype(i32) * np.int32(8) + t)
                pltpu.sync_copy(fbuf, o2.at[oidx])

    return gather


def _edge_body(wa_ref, x_ref, o_ref):
    x = x_ref[...]
    logit = jnp.zeros(x.shape[1:], f32)
    for k in range(8):
        logit = logit + wa_ref[0, k] * jax.nn.relu(x[k])
    alpha = 1.0 / (1.0 + jnp.exp(-logit))
    fs, ns2, gr, nr2 = x[8], x[9], x[10], x[11]
    x2 = fs * fs * ns2
    y2 = gr * gr * nr2
    t = 1.0 + C * y2
    den = jnp.maximum(1.0 + C * C * x2 * y2, MIN_NORM)
    aa = t / den * fs
    bb = (1.0 - C * x2) / den * gr
    m2 = aa * aa * ns2 + bb * bb * nr2
    pn = jnp.maximum(jnp.sqrt(m2), MIN_NORM)
    pf = jnp.where(pn > MAXNORM, MAXNORM / pn, 1.0)
    scale = _logmap0_scale(pf * pf * m2) * pf * alpha
    o_ref[0] = scale * aa
    o_ref[1] = scale * bb


def _edge_call(wa, o3, e_pad):
    nrow = e_pad // CHUNK
    br = 256
    nblk = nrow // br
    return pl.pallas_call(
        _edge_body,
        grid=(nblk,),
        in_specs=[
            pl.BlockSpec(memory_space=pltpu.SMEM),
            pl.BlockSpec((16, br, CHUNK), lambda i: (0, i, 0)),
        ],
        out_specs=pl.BlockSpec((2, br, CHUNK), lambda i: (0, i, 0)),
        out_shape=jax.ShapeDtypeStruct((2, nrow, CHUNK), f32),
    )(wa, o3)


def _make_scatter(e_pad):
    per_tile = e_pad // NW
    nchunk = per_tile // CHUNK
    rows_per_tile = A_PAD // NS
    mesh = plsc.VectorSubcoreMesh(core_axis_name="c", subcore_axis_name="s")

    @functools.partial(
        pl.kernel,
        out_type=jax.ShapeDtypeStruct((NC, NS, rows_per_tile, IN_DIM), f32),
        mesh=mesh,
        compiler_params=_SC_PARAMS,
        scratch_types=[
            pltpu.VMEM((CHUNK,), i32),
            pltpu.VMEM((CHUNK,), i32),
            pltpu.VMEM((CHUNK,), i32),
            pltpu.VMEM((CHUNK,), i32),
            pltpu.VMEM((CHUNK,), i32),
            pltpu.VMEM((CHUNK,), f32),
            pltpu.VMEM((CHUNK,), f32),
            pltpu.VMEM((CHUNK, IN_DIM), f32),
            pltpu.VMEM((CHUNK, IN_DIM), f32),
            pltpu.VMEM((CHUNK, IN_DIM), f32),
            pltpu.VMEM_SHARED((A_PAD, IN_DIM), f32),
            pltpu.SemaphoreType.DMA,
        ],
    )
    def scatter(hid, rela, psr, obj, w1f, w2f, out,
                eidx_v, psr_v, obj_v, sub_v, rel_v, w1_v, w2_v,
                hs_b, hr_b, msg_b, acc, sem):
        sid = lax.axis_index("s").astype(i32)
        cid = lax.axis_index("c").astype(i32)
        wid = sid * np.int32(NC) + cid
        base = wid * np.int32(per_tile)
        row0 = sid * np.int32(rows_per_tile)
        ii = lax.iota(i32, 16)

        zv = jnp.zeros((16,), f32)
        for e in range(CHUNK):
            for g in range(IN_DIM // 16):
                msg_b[e, pl.ds(g * 16, 16)] = zv

        @pl.loop(np.int32(0), np.int32(4), step=np.int32(1))
        def _zero(k):
            r = row0 + k.astype(i32) * np.int32(CHUNK)
            pltpu.sync_copy(msg_b, acc.at[pl.ds(r, CHUNK)])
        pltpu.sync_copy(msg_b.at[pl.ds(0, rows_per_tile - 4 * CHUNK)],
                        acc.at[pl.ds(row0 + np.int32(4 * CHUNK),
                                     rows_per_tile - 4 * CHUNK)])

        plsc.subcore_barrier()

        @pl.loop(np.int32(0), np.int32(nchunk), step=np.int32(1))
        def _chunk(j):
            off = base + j.astype(i32) * np.int32(CHUNK)
            for g in range(CHUNK // 16):
                eidx_v[pl.ds(g * 16, 16)] = ii + (off + np.int32(g * 16))
            c1 = pltpu.async_copy(psr.at[eidx_v], psr_v, sem)
            c2 = pltpu.async_copy(obj.at[eidx_v], obj_v, sem)
            c3 = pltpu.async_copy(w1f.at[eidx_v], w1_v, sem)
            c4 = pltpu.async_copy(w2f.at[eidx_v], w2_v, sem)
            c1.wait(); c2.wait(); c3.wait(); c4.wait()
            for g in range(CHUNK // 16):
                sl = pl.ds(g * 16, 16)
                p = psr_v[sl]
                sub_v[sl] = lax.shift_right_logical(p, 9)
                rel_v[sl] = lax.bitwise_and(p, np.int32(511))
            g1 = pltpu.async_copy(hid.at[sub_v], hs_b, sem)
            g2 = pltpu.async_copy(rela.at[rel_v], hr_b, sem)
            g1.wait(); g2.wait()

            @pl.loop(np.int32(0), np.int32(CHUNK), step=np.int32(1))
            def _edge(e):
                ee = jnp.full((16,), e, dtype=i32)
                w1v = plsc.load_gather(w1_v, [ee])
                w2v = plsc.load_gather(w2_v, [ee])
                for k in range(IN_DIM // 16):
                    sl = pl.ds(k * 16, 16)
                    msg_b[e, sl] = w1v * hs_b[e, sl] + w2v * hr_b[e, sl]

            pltpu.sync_copy(msg_b, acc.at[obj_v], add=True)

        plsc.subcore_barrier()
        pltpu.sync_copy(acc.at[pl.ds(row0, rows_per_tile)], out.at[cid].at[sid])

    return scatter


def _final_body(p_ref, wh_ref, o_ref):
    g = p_ref[0] + p_ref[1]
    a = lax.dot_general(g, wh_ref[...], (((1,), (1,)), ((), ())),
                        preferred_element_type=f32)
    an2 = jnp.sum(a * a, axis=-1, keepdims=True)
    fa = _expmap0_scale(an2)
    y = fa * a
    o_ref[...] = y * _logmap0_scale(fa * fa * an2)


def _final_call(parts, w_h):
    blk = 632
    nblk = 10112 // blk
    return pl.pallas_call(
        _final_body,
        grid=(nblk,),
        in_specs=[
            pl.BlockSpec((NC, blk, IN_DIM), lambda i: (0, i, 0)),
            pl.BlockSpec((IN_DIM, IN_DIM), lambda i: (0, 0)),
        ],
        out_specs=pl.BlockSpec((blk, IN_DIM), lambda i: (i, 0)),
        out_shape=jax.ShapeDtypeStruct((10112, IN_DIM), f32),
    )(parts, w_h)


def kernel(q_sub, q_rel, hidden, edges, n_node, old_nodes_new_idx, rela_embed,
           Ws_attn_W, Wr_attn_W, Wqr_attn_W, Wqr_attn_b, W_attn_W, W_h_W):
    with jax.enable_x64(False):
        return _run(q_sub, q_rel, hidden, edges, n_node, old_nodes_new_idx,
                    rela_embed, Ws_attn_W, Wr_attn_W, Wqr_attn_W, Wqr_attn_b,
                    W_attn_W, W_h_W)


def _run(q_sub, q_rel, hidden, edges, n_node, old_nodes_new_idx, rela_embed,
         Ws_attn_W, Wr_attn_W, Wqr_attn_W, Wqr_attn_b, W_attn_W, W_h_W):
    hidden = hidden.astype(f32)
    rela = rela_embed.astype(f32)
    n_hid = hidden.shape[0]
    n_rel = rela.shape[0]
    n_edge = edges.shape[0]
    e_pad = ((n_edge + NW * CHUNK * 8 - 1) // (NW * CHUNK * 8)) * (NW * CHUNK * 8)

    h_pad = jnp.pad(hidden, ((0, N_PAD - n_hid), (0, 0)))
    rela_pad = jnp.pad(rela, ((0, R_PAD - n_rel), (0, 0)))
    ws_pad = jnp.pad(Ws_attn_W.astype(f32).T, ((0, 0), (0, 8)))
    wr_pad = jnp.pad(Wr_attn_W.astype(f32).T, ((0, 0), (0, 8)))
    wqr_pad = jnp.pad(Wqr_attn_W.astype(f32).T, ((0, 0), (0, 8)))
    b_pad = jnp.pad(Wqr_attn_b.astype(f32), (0, 8)).reshape(1, 16)
    wa = W_attn_W.astype(f32)
    q_rel32 = q_rel.astype(i32).reshape(512, 1)

    npad = e_pad - n_edge
    sub_c = edges[:, 4].astype(i32)
    rel_c = edges[:, 2].astype(i32)
    psr = jnp.concatenate([sub_c * R_PAD + rel_c, jnp.zeros((npad,), i32)])
    ridx = jnp.concatenate([edges[:, 0].astype(i32), jnp.zeros((npad,), i32)])
    obj = jnp.concatenate([
        (edges[:, 5] + (n_node - n_hid)).astype(i32),
        jnp.full((npad,), N_NODE, i32),
    ])

    nrec = _prep_call(h_pad, ws_pad)
    rrec, qrec = _relprep_call(rela_pad, wr_pad, wqr_pad, b_pad, q_rel32)

    e3 = jnp.stack([
        lax.bitcast_convert_type(psr, f32),
        lax.bitcast_convert_type(ridx, f32),
    ], axis=1).reshape(e_pad // 64, CHUNK)
    o2 = _make_gather(e_pad)(
        nrec.reshape(-1), rrec.reshape(-1), qrec.reshape(-1), e3)

    nrow = e_pad // CHUNK
    w2d = _edge_call(wa, o2.reshape(16, nrow, CHUNK), e_pad)

    parts = _make_scatter(e_pad)(
        hidden, rela, psr, obj, w2d[0].reshape(-1), w2d[1].reshape(-1))

    parts = jnp.pad(parts.reshape(NC, A_PAD, IN_DIM),
                    ((0, 0), (0, 10112 - A_PAD), (0, 0)))
    out = _final_call(parts, W_h_W.astype(f32))
    return out[:n_hid]


# speedup vs baseline: 1.0393x; 1.0393x over previous
import functools

import jax
import jax.numpy as jnp
import numpy as np
from jax import lax
from jax.experimental import pallas as pl
from jax.experimental.pallas import tpu as pltpu
from jax.experimental.pallas import tpu_sc as plsc

f32 = jnp.float32
i32 = jnp.int32

MIN_NORM = 1e-15
C = float(np.float32(1e-6))
SQRT_C = float(np.sqrt(np.float32(1e-6)))
MAXNORM = float(np.float32(1.0 - 0.004) / np.float32(SQRT_C))

N_NODE = 10000
N_PAD = 10240
A_PAD = 10112
R_PAD = 512
IN_DIM = 128
NC, NS = 2, 16
NW = NC * NS
CHUNK = 128

_SC_PARAMS = pltpu.CompilerParams(needs_layout_passes=False)


def _expmap0_scale(ns2):
    ns = jnp.sqrt(ns2)
    un = jnp.maximum(ns, MIN_NORM)
    arg = jnp.clip(SQRT_C * un, -15.0, 15.0)
    g = jnp.tanh(arg) / (SQRT_C * un)
    gn = jnp.maximum(g * ns, MIN_NORM)
    return g * jnp.where(gn > MAXNORM, MAXNORM / gn, 1.0)


def _logmap0_scale(yn2):
    yn = jnp.maximum(jnp.sqrt(yn2), MIN_NORM)
    z = jnp.clip(SQRT_C * yn, -1 + 1e-05, 1 - 1e-05)
    art = 0.5 * jnp.log((1.0 + z) / (1.0 - z))
    return art / (yn * SQRT_C)


def _prep_body(h_ref, ws_ref, rec_ref):
    h = h_ref[...]
    a = jnp.dot(h, ws_ref[...], preferred_element_type=f32)
    ns2 = jnp.sum(h * h, axis=-1, keepdims=True)
    fs = _expmap0_scale(ns2)
    lane = lax.broadcasted_iota(i32, (1, 16), 1)
    rec = a + jnp.where(lane == 8, fs, 0.0) + jnp.where(lane == 9, ns2, 0.0)
    rec_ref[...] = lax.slice(rec, (0, 0), (rec.shape[0], 10))


def _prep_call(h_pad, ws_pad):
    nblk = N_PAD // 512
    return pl.pallas_call(
        _prep_body,
        grid=(nblk,),
        in_specs=[
            pl.BlockSpec((512, IN_DIM), lambda i: (i, 0)),
            pl.BlockSpec((IN_DIM, 16), lambda i: (0, 0)),
        ],
        out_specs=pl.BlockSpec((512, 10), lambda i: (i, 0)),
        out_shape=jax.ShapeDtypeStruct((N_PAD, 10), f32),
    )(h_pad, ws_pad)


def _relprep_body(rp_ref, wr_ref, wqr_ref, b_ref, q_ref, rrec_ref, qrec_ref):
    rp = rp_ref[...]
    a = jnp.dot(rp, wr_ref[...], preferred_element_type=f32)
    nr2 = jnp.sum(rp * rp, axis=-1, keepdims=True)
    gr = _expmap0_scale(nr2)
    lane = lax.broadcasted_iota(i32, (1, 16), 1)
    rec = a + jnp.where(lane == 8, gr, 0.0) + jnp.where(lane == 9, nr2, 0.0)
    rrec_ref[...] = lax.slice(rec, (0, 0), (512, 10))
    col = lax.broadcasted_iota(i32, (512, R_PAD), 1)
    oh = jnp.where(col == q_ref[...], 1.0, 0.0).astype(f32)
    hq = jnp.dot(oh, rp, preferred_element_type=f32)
    qrec = jnp.dot(hq, wqr_ref[...], preferred_element_type=f32) + b_ref[...]
    qrec_ref[...] = lax.slice(qrec, (0, 0), (512, 8))


def _relprep_call(rela_pad, wr_pad, wqr_pad, b_pad, q_rel32):
    return pl.pallas_call(
        _relprep_body,
        in_specs=[
            pl.BlockSpec((R_PAD, IN_DIM), lambda: (0, 0)),
            pl.BlockSpec((IN_DIM, 16), lambda: (0, 0)),
            pl.BlockSpec((IN_DIM, 16), lambda: (0, 0)),
            pl.BlockSpec((1, 16), lambda: (0, 0)),
            pl.BlockSpec((512, 1), lambda: (0, 0)),
        ],
        out_specs=[
            pl.BlockSpec((R_PAD, 10), lambda: (0, 0)),
            pl.BlockSpec((512, 8), lambda: (0, 0)),
        ],
        out_shape=[
            jax.ShapeDtypeStruct((R_PAD, 10), f32),
            jax.ShapeDtypeStruct((512, 8), f32),
        ],
    )(rela_pad, wr_pad, wqr_pad, b_pad, q_rel32)


def _make_gather(e_pad):
    per_tile = e_pad // NW
    nchunk = per_tile // CHUNK
    nrow = e_pad // CHUNK
    mesh = plsc.VectorSubcoreMesh(core_axis_name="c", subcore_axis_name="s")

    @functools.partial(
        pl.kernel,
        out_type=jax.ShapeDtypeStruct((16 * nrow, CHUNK), f32),
        mesh=mesh,
        compiler_params=_SC_PARAMS,
        scratch_types=[
            pltpu.VMEM((N_PAD * 10,), f32),
            pltpu.VMEM((R_PAD * 10,), f32),
            pltpu.VMEM((512 * 8,), f32),
            pltpu.VMEM((CHUNK,), i32),
            pltpu.VMEM((CHUNK,), i32),
            pltpu.VMEM((CHUNK,), i32),
            pltpu.VMEM((16, CHUNK), f32),
            pltpu.SemaphoreType.DMA,
        ],
    )
    def gather(nflat, rflat, qflat, psr, ridx, o2,
               nrec_t, rrec_t, qrec_t, eidx_v, psr_v, ridx_v, fbuf, sem):
        sid = lax.axis_index("s").astype(i32)
        cid = lax.axis_index("c").astype(i32)
        wid = sid * np.int32(NC) + cid
        base = wid * np.int32(per_tile)
        rbase = wid * np.int32(nchunk)

        pltpu.sync_copy(nflat, nrec_t)
        pltpu.sync_copy(rflat, rrec_t)
        pltpu.sync_copy(qflat, qrec_t)

        zv = jnp.zeros((16,), f32)
        for r in range(12, 16):
            for g in range(CHUNK // 16):
                fbuf[r, pl.ds(g * 16, 16)] = zv

        ii = lax.iota(i32, 16)

        @pl.loop(np.int32(0), np.int32(nchunk), step=np.int32(1))
        def _chunk(j):
            jj = j.astype(i32)
            off = base + jj * np.int32(CHUNK)
            for g in range(CHUNK // 16):
                eidx_v[pl.ds(g * 16, 16)] = ii + (off + np.int32(g * 16))
            c1 = pltpu.async_copy(psr.at[eidx_v], psr_v, sem)
            c2 = pltpu.async_copy(ridx.at[eidx_v], ridx_v, sem)
            c1.wait(); c2.wait()
            for g in range(CHUNK // 16):
                sl = pl.ds(g * 16, 16)
                p = psr_v[sl]
                sub = lax.shift_right_logical(p, 9)
                rel = lax.bitwise_and(p, np.int32(511))
                s10 = sub * np.int32(10)
                r10 = rel * np.int32(10)
                q8 = ridx_v[sl] * np.int32(8)
                for k in range(8):
                    v = (plsc.load_gather(nrec_t, [s10 + np.int32(k)])
                         + plsc.load_gather(rrec_t, [r10 + np.int32(k)])
                         + plsc.load_gather(qrec_t, [q8 + np.int32(k)]))
                    fbuf[k, sl] = v
                fbuf[8, sl] = plsc.load_gather(nrec_t, [s10 + np.int32(8)])
                fbuf[9, sl] = plsc.load_gather(nrec_t, [s10 + np.int32(9)])
                fbuf[10, sl] = plsc.load_gather(rrec_t, [r10 + np.int32(8)])
                fbuf[11, sl] = plsc.load_gather(rrec_t, [r10 + np.int32(9)])
            oidx = ii * np.int32(nrow) + (rbase + jj)
            pltpu.sync_copy(fbuf, o2.at[oidx])

    return gather


def _edge_body(wa_ref, x_ref, o_ref):
    x = x_ref[...]
    logit = jnp.zeros(x.shape[1:], f32)
    for k in range(8):
        logit = logit + wa_ref[0, k] * jax.nn.relu(x[k])
    alpha = 1.0 / (1.0 + jnp.exp(-logit))
    fs, ns2, gr, nr2 = x[8], x[9], x[10], x[11]
    x2 = fs * fs * ns2
    y2 = gr * gr * nr2
    t = 1.0 + C * y2
    den = jnp.maximum(1.0 + C * C * x2 * y2, MIN_NORM)
    aa = t / den * fs
    bb = (1.0 - C * x2) / den * gr
    m2 = aa * aa * ns2 + bb * bb * nr2
    pn = jnp.maximum(jnp.sqrt(m2), MIN_NORM)
    pf = jnp.where(pn > MAXNORM, MAXNORM / pn, 1.0)
    scale = _logmap0_scale(pf * pf * m2) * pf * alpha
    o_ref[0] = scale * aa
    o_ref[1] = scale * bb


def _edge_call(wa, o3, e_pad):
    nrow = e_pad // CHUNK
    br = 256
    nblk = nrow // br
    return pl.pallas_call(
        _edge_body,
        grid=(nblk,),
        in_specs=[
            pl.BlockSpec(memory_space=pltpu.SMEM),
            pl.BlockSpec((16, br, CHUNK), lambda i: (0, i, 0)),
        ],
        out_specs=pl.BlockSpec((2, br, CHUNK), lambda i: (0, i, 0)),
        out_shape=jax.ShapeDtypeStruct((2, nrow, CHUNK), f32),
    )(wa, o3)


def _make_scatter(e_pad):
    per_tile = e_pad // NW
    nchunk = per_tile // CHUNK
    rows_per_tile = A_PAD // NS
    mesh = plsc.VectorSubcoreMesh(core_axis_name="c", subcore_axis_name="s")

    @functools.partial(
        pl.kernel,
        out_type=jax.ShapeDtypeStruct((NC, NS, rows_per_tile, IN_DIM), f32),
        mesh=mesh,
        compiler_params=_SC_PARAMS,
        scratch_types=[
            pltpu.VMEM((CHUNK,), i32),
            pltpu.VMEM((CHUNK,), i32),
            pltpu.VMEM((CHUNK,), i32),
            pltpu.VMEM((CHUNK,), i32),
            pltpu.VMEM((CHUNK,), i32),
            pltpu.VMEM((CHUNK,), f32),
            pltpu.VMEM((CHUNK,), f32),
            pltpu.VMEM((CHUNK, IN_DIM), f32),
            pltpu.VMEM((CHUNK, IN_DIM), f32),
            pltpu.VMEM((CHUNK, IN_DIM), f32),
            pltpu.VMEM_SHARED((A_PAD, IN_DIM), f32),
            pltpu.SemaphoreType.DMA,
            pltpu.SemaphoreType.DMA,
        ],
    )
    def scatter(hid, rela, psr, obj, w1f, w2f, dmy_f, out,
                eidx_v, psr_v, obj_v, sub_v, rel_v, w1_v, w2_v,
                hs_b, hr_b, msg_b, acc, sem, sem_s):
        sid = lax.axis_index("s").astype(i32)
        cid = lax.axis_index("c").astype(i32)
        wid = sid * np.int32(NC) + cid
        base = wid * np.int32(per_tile)
        row0 = sid * np.int32(rows_per_tile)
        ii = lax.iota(i32, 16)

        zv = jnp.zeros((16,), f32)
        for e in range(CHUNK):
            for g in range(IN_DIM // 16):
                msg_b[e, pl.ds(g * 16, 16)] = zv

        @pl.loop(np.int32(0), np.int32(4), step=np.int32(1))
        def _zero(k):
            r = row0 + k.astype(i32) * np.int32(CHUNK)
            pltpu.sync_copy(msg_b, acc.at[pl.ds(r, CHUNK)])
        pltpu.sync_copy(msg_b.at[pl.ds(0, rows_per_tile - 4 * CHUNK)],
                        acc.at[pl.ds(row0 + np.int32(4 * CHUNK),
                                     rows_per_tile - 4 * CHUNK)])

        plsc.subcore_barrier()

        @pl.loop(np.int32(0), np.int32(nchunk), step=np.int32(1))
        def _chunk(j):
            off = base + j.astype(i32) * np.int32(CHUNK)

            @pl.when(j.astype(i32) >= np.int32(1))
            def _drain():
                pltpu.make_async_copy(dmy_f, msg_b, sem_s).wait()

            for g in range(CHUNK // 16):
                eidx_v[pl.ds(g * 16, 16)] = ii + (off + np.int32(g * 16))
            c1 = pltpu.async_copy(psr.at[eidx_v], psr_v, sem)
            c2 = pltpu.async_copy(obj.at[eidx_v], obj_v, sem)
            c3 = pltpu.async_copy(w1f.at[eidx_v], w1_v, sem)
            c4 = pltpu.async_copy(w2f.at[eidx_v], w2_v, sem)
            c1.wait(); c2.wait(); c3.wait(); c4.wait()
            for g in range(CHUNK // 16):
                sl = pl.ds(g * 16, 16)
                p = psr_v[sl]
                sub_v[sl] = lax.shift_right_logical(p, 9)
                rel_v[sl] = lax.bitwise_and(p, np.int32(511))
            g1 = pltpu.async_copy(hid.at[sub_v], hs_b, sem)
            g2 = pltpu.async_copy(rela.at[rel_v], hr_b, sem)
            g1.wait(); g2.wait()

            @pl.loop(np.int32(0), np.int32(CHUNK), step=np.int32(1))
            def _edge(e):
                ee = jnp.full((16,), e, dtype=i32)
                w1v = plsc.load_gather(w1_v, [ee])
                w2v = plsc.load_gather(w2_v, [ee])
                for k in range(IN_DIM // 16):
                    sl = pl.ds(k * 16, 16)
                    msg_b[e, sl] = w1v * hs_b[e, sl] + w2v * hr_b[e, sl]

            pltpu.async_copy(msg_b, acc.at[obj_v], sem_s, add=True)

        pltpu.make_async_copy(dmy_f, msg_b, sem_s).wait()
        plsc.subcore_barrier()
        pltpu.sync_copy(acc.at[pl.ds(row0, rows_per_tile)], out.at[cid].at[sid])

    return scatter


def _final_body(p_ref, wh_ref, o_ref):
    g = p_ref[0] + p_ref[1]
    a = lax.dot_general(g, wh_ref[...], (((1,), (1,)), ((), ())),
                        preferred_element_type=f32)
    an2 = jnp.sum(a * a, axis=-1, keepdims=True)
    fa = _expmap0_scale(an2)
    y = fa * a
    o_ref[...] = y * _logmap0_scale(fa * fa * an2)


def _final_call(parts, w_h):
    blk = 632
    nblk = 10112 // blk
    return pl.pallas_call(
        _final_body,
        grid=(nblk,),
        in_specs=[
            pl.BlockSpec((NC, blk, IN_DIM), lambda i: (0, i, 0)),
            pl.BlockSpec((IN_DIM, IN_DIM), lambda i: (0, 0)),
        ],
        out_specs=pl.BlockSpec((blk, IN_DIM), lambda i: (i, 0)),
        out_shape=jax.ShapeDtypeStruct((10112, IN_DIM), f32),
    )(parts, w_h)


def kernel(q_sub, q_rel, hidden, edges, n_node, old_nodes_new_idx, rela_embed,
           Ws_attn_W, Wr_attn_W, Wqr_attn_W, Wqr_attn_b, W_attn_W, W_h_W):
    with jax.enable_x64(False):
        return _run(q_sub, q_rel, hidden, edges, n_node, old_nodes_new_idx,
                    rela_embed, Ws_attn_W, Wr_attn_W, Wqr_attn_W, Wqr_attn_b,
                    W_attn_W, W_h_W)


def _run(q_sub, q_rel, hidden, edges, n_node, old_nodes_new_idx, rela_embed,
         Ws_attn_W, Wr_attn_W, Wqr_attn_W, Wqr_attn_b, W_attn_W, W_h_W):
    hidden = hidden.astype(f32)
    rela = rela_embed.astype(f32)
    n_hid = hidden.shape[0]
    n_rel = rela.shape[0]
    n_edge = edges.shape[0]
    e_pad = ((n_edge + NW * CHUNK * 8 - 1) // (NW * CHUNK * 8)) * (NW * CHUNK * 8)

    h_pad = jnp.pad(hidden, ((0, N_PAD - n_hid), (0, 0)))
    rela_pad = jnp.pad(rela, ((0, R_PAD - n_rel), (0, 0)))
    ws_pad = jnp.pad(Ws_attn_W.astype(f32).T, ((0, 0), (0, 8)))
    wr_pad = jnp.pad(Wr_attn_W.astype(f32).T, ((0, 0), (0, 8)))
    wqr_pad = jnp.pad(Wqr_attn_W.astype(f32).T, ((0, 0), (0, 8)))
    b_pad = jnp.pad(Wqr_attn_b.astype(f32), (0, 8)).reshape(1, 16)
    wa = W_attn_W.astype(f32)
    q_rel32 = q_rel.astype(i32).reshape(512, 1)

    npad = e_pad - n_edge
    sub_c = edges[:, 4].astype(i32)
    rel_c = edges[:, 2].astype(i32)
    psr = jnp.concatenate([sub_c * R_PAD + rel_c, jnp.zeros((npad,), i32)])
    ridx = jnp.concatenate([edges[:, 0].astype(i32), jnp.zeros((npad,), i32)])
    obj = jnp.concatenate([
        (edges[:, 5] + (n_node - n_hid)).astype(i32),
        jnp.full((npad,), N_NODE, i32),
    ])

    nrec = _prep_call(h_pad, ws_pad)
    rrec, qrec = _relprep_call(rela_pad, wr_pad, wqr_pad, b_pad, q_rel32)

    o2 = _make_gather(e_pad)(
        nrec.reshape(-1), rrec.reshape(-1), qrec.reshape(-1), psr, ridx)

    nrow = e_pad // CHUNK
    w2d = _edge_call(wa, o2.reshape(16, nrow, CHUNK), e_pad)

    dmy_f = jnp.zeros((CHUNK, IN_DIM), f32)
    parts = _make_scatter(e_pad)(
        hidden, rela, psr, obj, w2d[0].reshape(-1), w2d[1].reshape(-1), dmy_f)

    parts = jnp.pad(parts.reshape(NC, A_PAD, IN_DIM),
                    ((0, 0), (0, 10112 - A_PAD), (0, 0)))
    out = _final_call(parts, W_h_W.astype(f32))
    return out[:n_hid]


# speedup vs baseline: 1.0424x; 1.0030x over previous
import functools

import jax
import jax.numpy as jnp
import numpy as np
from jax import lax
from jax.experimental import pallas as pl
from jax.experimental.pallas import tpu as pltpu
from jax.experimental.pallas import tpu_sc as plsc

f32 = jnp.float32
i32 = jnp.int32

MIN_NORM = 1e-15
C = float(np.float32(1e-6))
SQRT_C = float(np.sqrt(np.float32(1e-6)))
MAXNORM = float(np.float32(1.0 - 0.004) / np.float32(SQRT_C))

N_NODE = 10000
N_PAD = 10240
A_PAD = 10112
R_PAD = 512
IN_DIM = 128
NC, NS = 2, 16
NW = NC * NS
CHUNK = 128

_SC_PARAMS = pltpu.CompilerParams(needs_layout_passes=False)


def _expmap0_scale(ns2):
    ns = jnp.sqrt(ns2)
    un = jnp.maximum(ns, MIN_NORM)
    arg = jnp.clip(SQRT_C * un, -15.0, 15.0)
    g = jnp.tanh(arg) / (SQRT_C * un)
    gn = jnp.maximum(g * ns, MIN_NORM)
    return g * jnp.where(gn > MAXNORM, MAXNORM / gn, 1.0)


def _logmap0_scale(yn2):
    yn = jnp.maximum(jnp.sqrt(yn2), MIN_NORM)
    z = jnp.clip(SQRT_C * yn, -1 + 1e-05, 1 - 1e-05)
    art = 0.5 * jnp.log((1.0 + z) / (1.0 - z))
    return art / (yn * SQRT_C)


def _prep_body(h_ref, ws_ref, rec_ref):
    h = h_ref[...]
    a = jnp.dot(h, ws_ref[...], preferred_element_type=f32)
    ns2 = jnp.sum(h * h, axis=-1, keepdims=True)
    fs = _expmap0_scale(ns2)
    lane = lax.broadcasted_iota(i32, (1, 16), 1)
    rec = a + jnp.where(lane == 8, fs, 0.0) + jnp.where(lane == 9, ns2, 0.0)
    rec_ref[...] = lax.slice(rec, (0, 0), (rec.shape[0], 10))


def _prep_call(h_pad, ws_pad):
    nblk = N_PAD // 512
    return pl.pallas_call(
        _prep_body,
        grid=(nblk,),
        in_specs=[
            pl.BlockSpec((512, IN_DIM), lambda i: (i, 0)),
            pl.BlockSpec((IN_DIM, 16), lambda i: (0, 0)),
        ],
        out_specs=pl.BlockSpec((512, 10), lambda i: (i, 0)),
        out_shape=jax.ShapeDtypeStruct((N_PAD, 10), f32),
    )(h_pad, ws_pad)


def _relprep_body(rp_ref, wr_ref, wqr_ref, b_ref, q_ref, rrec_ref, qrec_ref):
    rp = rp_ref[...]
    a = jnp.dot(rp, wr_ref[...], preferred_element_type=f32)
    nr2 = jnp.sum(rp * rp, axis=-1, keepdims=True)
    gr = _expmap0_scale(nr2)
    lane = lax.broadcasted_iota(i32, (1, 16), 1)
    rec = a + jnp.where(lane == 8, gr, 0.0) + jnp.where(lane == 9, nr2, 0.0)
    rrec_ref[...] = lax.slice(rec, (0, 0), (512, 10))
    col = lax.broadcasted_iota(i32, (512, R_PAD), 1)
    oh = jnp.where(col == q_ref[...], 1.0, 0.0).astype(f32)
    hq = jnp.dot(oh, rp, preferred_element_type=f32)
    qrec = jnp.dot(hq, wqr_ref[...], preferred_element_type=f32) + b_ref[...]
    qrec_ref[...] = lax.slice(qrec, (0, 0), (512, 8))


def _relprep_call(rela_pad, wr_pad, wqr_pad, b_pad, q_rel32):
    return pl.pallas_call(
        _relprep_body,
        in_specs=[
            pl.BlockSpec((R_PAD, IN_DIM), lambda: (0, 0)),
            pl.BlockSpec((IN_DIM, 16), lambda: (0, 0)),
            pl.BlockSpec((IN_DIM, 16), lambda: (0, 0)),
            pl.BlockSpec((1, 16), lambda: (0, 0)),
            pl.BlockSpec((512, 1), lambda: (0, 0)),
        ],
        out_specs=[
            pl.BlockSpec((R_PAD, 10), lambda: (0, 0)),
            pl.BlockSpec((512, 8), lambda: (0, 0)),
        ],
        out_shape=[
            jax.ShapeDtypeStruct((R_PAD, 10), f32),
            jax.ShapeDtypeStruct((512, 8), f32),
        ],
    )(rela_pad, wr_pad, wqr_pad, b_pad, q_rel32)


def _make_gather(e_pad):
    per_tile = e_pad // NW
    nchunk = per_tile // CHUNK
    nrow = e_pad // CHUNK
    mesh = plsc.VectorSubcoreMesh(core_axis_name="c", subcore_axis_name="s")

    @functools.partial(
        pl.kernel,
        out_type=jax.ShapeDtypeStruct((16 * nrow, CHUNK), f32),
        mesh=mesh,
        compiler_params=_SC_PARAMS,
        scratch_types=[
            pltpu.VMEM((N_PAD * 10,), f32),
            pltpu.VMEM((R_PAD * 10,), f32),
            pltpu.VMEM((512 * 8,), f32),
            pltpu.VMEM((CHUNK,), i32),
            pltpu.VMEM((CHUNK,), i32),
            pltpu.VMEM((CHUNK,), i32),
            pltpu.VMEM((16, CHUNK), f32),
            pltpu.SemaphoreType.DMA,
        ],
    )
    def gather(nflat, rflat, qflat, psr, ridx, o2,
               nrec_t, rrec_t, qrec_t, eidx_v, psr_v, ridx_v, fbuf, sem):
        sid = lax.axis_index("s").astype(i32)
        cid = lax.axis_index("c").astype(i32)
        wid = sid * np.int32(NC) + cid
        base = wid * np.int32(per_tile)
        rbase = wid * np.int32(nchunk)

        pltpu.sync_copy(nflat, nrec_t)
        pltpu.sync_copy(rflat, rrec_t)
        pltpu.sync_copy(qflat, qrec_t)

        zv = jnp.zeros((16,), f32)
        for r in range(12, 16):
            for g in range(CHUNK // 16):
                fbuf[r, pl.ds(g * 16, 16)] = zv

        ii = lax.iota(i32, 16)

        @pl.loop(np.int32(0), np.int32(nchunk), step=np.int32(1))
        def _chunk(j):
            jj = j.astype(i32)
            off = base + jj * np.int32(CHUNK)
            for g in range(CHUNK // 16):
                eidx_v[pl.ds(g * 16, 16)] = ii + (off + np.int32(g * 16))
            c1 = pltpu.async_copy(psr.at[eidx_v], psr_v, sem)
            c2 = pltpu.async_copy(ridx.at[eidx_v], ridx_v, sem)
            c1.wait(); c2.wait()
            for g in range(CHUNK // 16):
                sl = pl.ds(g * 16, 16)
                p = psr_v[sl]
                sub = lax.shift_right_logical(p, 9)
                rel = lax.bitwise_and(p, np.int32(511))
                s10 = sub * np.int32(10)
                r10 = rel * np.int32(10)
                q8 = ridx_v[sl] * np.int32(8)
                for k in range(8):
                    v = (plsc.load_gather(nrec_t, [s10 + np.int32(k)])
                         + plsc.load_gather(rrec_t, [r10 + np.int32(k)])
                         + plsc.load_gather(qrec_t, [q8 + np.int32(k)]))
                    fbuf[k, sl] = v
                fbuf[8, sl] = plsc.load_gather(nrec_t, [s10 + np.int32(8)])
                fbuf[9, sl] = plsc.load_gather(nrec_t, [s10 + np.int32(9)])
                fbuf[10, sl] = plsc.load_gather(rrec_t, [r10 + np.int32(8)])
                fbuf[11, sl] = plsc.load_gather(rrec_t, [r10 + np.int32(9)])
            oidx = ii * np.int32(nrow) + (rbase + jj)
            pltpu.sync_copy(fbuf, o2.at[oidx])

    return gather


def _edge_body(wa_ref, x_ref, o_ref):
    x = x_ref[...]
    logit = jnp.zeros(x.shape[1:], f32)
    for k in range(8):
        logit = logit + wa_ref[0, k] * jax.nn.relu(x[k])
    alpha = 1.0 / (1.0 + jnp.exp(-logit))
    fs, ns2, gr, nr2 = x[8], x[9], x[10], x[11]
    x2 = fs * fs * ns2
    y2 = gr * gr * nr2
    t = 1.0 + C * y2
    den = jnp.maximum(1.0 + C * C * x2 * y2, MIN_NORM)
    aa = t / den * fs
    bb = (1.0 - C * x2) / den * gr
    m2 = aa * aa * ns2 + bb * bb * nr2
    pn = jnp.maximum(jnp.sqrt(m2), MIN_NORM)
    pf = jnp.where(pn > MAXNORM, MAXNORM / pn, 1.0)
    scale = _logmap0_scale(pf * pf * m2) * pf * alpha
    o_ref[0] = scale * aa
    o_ref[1] = scale * bb


def _edge_call(wa, o3, e_pad):
    nrow = e_pad // CHUNK
    br = 256
    nblk = nrow // br
    return pl.pallas_call(
        _edge_body,
        grid=(nblk,),
        in_specs=[
            pl.BlockSpec(memory_space=pltpu.SMEM),
            pl.BlockSpec((16, br, CHUNK), lambda i: (0, i, 0)),
        ],
        out_specs=pl.BlockSpec((2, br, CHUNK), lambda i: (0, i, 0)),
        out_shape=jax.ShapeDtypeStruct((2, nrow, CHUNK), f32),
    )(wa, o3)


def _make_scatter(e_pad):
    per_tile = e_pad // NW
    nchunk = per_tile // CHUNK
    rows_per_tile = A_PAD // NS
    mesh = plsc.VectorSubcoreMesh(core_axis_name="c", subcore_axis_name="s")

    @functools.partial(
        pl.kernel,
        out_type=jax.ShapeDtypeStruct((NC, NS, rows_per_tile, IN_DIM), f32),
        mesh=mesh,
        compiler_params=_SC_PARAMS,
        scratch_types=[
            pltpu.VMEM((CHUNK,), i32),
            pltpu.VMEM((CHUNK,), i32),
            pltpu.VMEM((CHUNK,), i32),
            pltpu.VMEM((CHUNK,), i32),
            pltpu.VMEM((CHUNK,), i32),
            pltpu.VMEM((CHUNK,), f32),
            pltpu.VMEM((CHUNK,), f32),
            pltpu.VMEM((CHUNK, IN_DIM), f32),
            pltpu.VMEM((CHUNK, IN_DIM), f32),
            pltpu.VMEM((CHUNK, IN_DIM), f32),
            pltpu.VMEM_SHARED((A_PAD, IN_DIM), f32),
            pltpu.SemaphoreType.DMA,
        ],
    )
    def scatter(hid, rela, psr, obj, w1f, w2f, out,
                eidx_v, psr_v, obj_v, sub_v, rel_v, w1_v, w2_v,
                hs_b, hr_b, msg_b, acc, sem):
        sid = lax.axis_index("s").astype(i32)
        cid = lax.axis_index("c").astype(i32)
        wid = sid * np.int32(NC) + cid
        base = wid * np.int32(per_tile)
        row0 = sid * np.int32(rows_per_tile)
        ii = lax.iota(i32, 16)

        zv = jnp.zeros((16,), f32)
        for e in range(CHUNK):
            for g in range(IN_DIM // 16):
                msg_b[e, pl.ds(g * 16, 16)] = zv

        @pl.loop(np.int32(0), np.int32(4), step=np.int32(1))
        def _zero(k):
            r = row0 + k.astype(i32) * np.int32(CHUNK)
            pltpu.sync_copy(msg_b, acc.at[pl.ds(r, CHUNK)])
        pltpu.sync_copy(msg_b.at[pl.ds(0, rows_per_tile - 4 * CHUNK)],
                        acc.at[pl.ds(row0 + np.int32(4 * CHUNK),
                                     rows_per_tile - 4 * CHUNK)])

        plsc.subcore_barrier()

        @pl.loop(np.int32(0), np.int32(nchunk), step=np.int32(1))
        def _chunk(j):
            off = base + j.astype(i32) * np.int32(CHUNK)
            for g in range(CHUNK // 16):
                eidx_v[pl.ds(g * 16, 16)] = ii + (off + np.int32(g * 16))
            c1 = pltpu.async_copy(psr.at[eidx_v], psr_v, sem)
            c2 = pltpu.async_copy(obj.at[eidx_v], obj_v, sem)
            c3 = pltpu.async_copy(w1f.at[eidx_v], w1_v, sem)
            c4 = pltpu.async_copy(w2f.at[eidx_v], w2_v, sem)
            c1.wait(); c2.wait(); c3.wait(); c4.wait()
            for g in range(CHUNK // 16):
                sl = pl.ds(g * 16, 16)
                p = psr_v[sl]
                sub_v[sl] = lax.shift_right_logical(p, 9)
                rel_v[sl] = lax.bitwise_and(p, np.int32(511))
            g1 = pltpu.async_copy(hid.at[sub_v], hs_b, sem)
            g2 = pltpu.async_copy(rela.at[rel_v], hr_b, sem)
            g1.wait(); g2.wait()

            @pl.loop(np.int32(0), np.int32(CHUNK), step=np.int32(1),
                     unroll=4)
            def _edge(e):
                ee = jnp.full((16,), e, dtype=i32)
                w1v = plsc.load_gather(w1_v, [ee])
                w2v = plsc.load_gather(w2_v, [ee])
                for k in range(IN_DIM // 16):
                    sl = pl.ds(k * 16, 16)
                    msg_b[e, sl] = w1v * hs_b[e, sl] + w2v * hr_b[e, sl]

            pltpu.sync_copy(msg_b, acc.at[obj_v], add=True)

        plsc.subcore_barrier()
        pltpu.sync_copy(acc.at[pl.ds(row0, rows_per_tile)], out.at[cid].at[sid])

    return scatter


def _final_body(p_ref, wh_ref, o_ref):
    g = p_ref[0] + p_ref[1]
    a = lax.dot_general(g, wh_ref[...], (((1,), (1,)), ((), ())),
                        preferred_element_type=f32)
    an2 = jnp.sum(a * a, axis=-1, keepdims=True)
    fa = _expmap0_scale(an2)
    y = fa * a
    o_ref[...] = y * _logmap0_scale(fa * fa * an2)


def _final_call(parts, w_h):
    blk = 632
    nblk = 10112 // blk
    return pl.pallas_call(
        _final_body,
        grid=(nblk,),
        in_specs=[
            pl.BlockSpec((NC, blk, IN_DIM), lambda i: (0, i, 0)),
            pl.BlockSpec((IN_DIM, IN_DIM), lambda i: (0, 0)),
        ],
        out_specs=pl.BlockSpec((blk, IN_DIM), lambda i: (i, 0)),
        out_shape=jax.ShapeDtypeStruct((10112, IN_DIM), f32),
    )(parts, w_h)


def kernel(q_sub, q_rel, hidden, edges, n_node, old_nodes_new_idx, rela_embed,
           Ws_attn_W, Wr_attn_W, Wqr_attn_W, Wqr_attn_b, W_attn_W, W_h_W):
    with jax.enable_x64(False):
        return _run(q_sub, q_rel, hidden, edges, n_node, old_nodes_new_idx,
                    rela_embed, Ws_attn_W, Wr_attn_W, Wqr_attn_W, Wqr_attn_b,
                    W_attn_W, W_h_W)


def _run(q_sub, q_rel, hidden, edges, n_node, old_nodes_new_idx, rela_embed,
         Ws_attn_W, Wr_attn_W, Wqr_attn_W, Wqr_attn_b, W_attn_W, W_h_W):
    hidden = hidden.astype(f32)
    rela = rela_embed.astype(f32)
    n_hid = hidden.shape[0]
    n_rel = rela.shape[0]
    n_edge = edges.shape[0]
    e_pad = ((n_edge + NW * CHUNK * 8 - 1) // (NW * CHUNK * 8)) * (NW * CHUNK * 8)

    h_pad = jnp.pad(hidden, ((0, N_PAD - n_hid), (0, 0)))
    rela_pad = jnp.pad(rela, ((0, R_PAD - n_rel), (0, 0)))
    ws_pad = jnp.pad(Ws_attn_W.astype(f32).T, ((0, 0), (0, 8)))
    wr_pad = jnp.pad(Wr_attn_W.astype(f32).T, ((0, 0), (0, 8)))
    wqr_pad = jnp.pad(Wqr_attn_W.astype(f32).T, ((0, 0), (0, 8)))
    b_pad = jnp.pad(Wqr_attn_b.astype(f32), (0, 8)).reshape(1, 16)
    wa = W_attn_W.astype(f32)
    q_rel32 = q_rel.astype(i32).reshape(512, 1)

    npad = e_pad - n_edge
    sub_c = edges[:, 4].astype(i32)
    rel_c = edges[:, 2].astype(i32)
    psr = jnp.concatenate([sub_c * R_PAD + rel_c, jnp.zeros((npad,), i32)])
    ridx = jnp.concatenate([edges[:, 0].astype(i32), jnp.zeros((npad,), i32)])
    obj = jnp.concatenate([
        (edges[:, 5] + (n_node - n_hid)).astype(i32),
        jnp.full((npad,), N_NODE, i32),
    ])

    nrec = _prep_call(h_pad, ws_pad)
    rrec, qrec = _relprep_call(rela_pad, wr_pad, wqr_pad, b_pad, q_rel32)

    o2 = _make_gather(e_pad)(
        nrec.reshape(-1), rrec.reshape(-1), qrec.reshape(-1), psr, ridx)

    nrow = e_pad // CHUNK
    w2d = _edge_call(wa, o2.reshape(16, nrow, CHUNK), e_pad)

    parts = _make_scatter(e_pad)(
        hidden, rela, psr, obj, w2d[0].reshape(-1), w2d[1].reshape(-1))

    parts = jnp.pad(parts.reshape(NC, A_PAD, IN_DIM),
                    ((0, 0), (0, 10112 - A_PAD), (0, 0)))
    out = _final_call(parts, W_h_W.astype(f32))
    return out[:n_hid]


# speedup vs baseline: 1.0478x; 1.0052x over previous
import functools

import jax
import jax.numpy as jnp
import numpy as np
from jax import lax
from jax.experimental import pallas as pl
from jax.experimental.pallas import tpu as pltpu
from jax.experimental.pallas import tpu_sc as plsc

f32 = jnp.float32
i32 = jnp.int32

MIN_NORM = 1e-15
C = float(np.float32(1e-6))
SQRT_C = float(np.sqrt(np.float32(1e-6)))
MAXNORM = float(np.float32(1.0 - 0.004) / np.float32(SQRT_C))

N_NODE = 10000
N_PAD = 10240
A_PAD = 10112
R_PAD = 512
IN_DIM = 128
NC, NS = 2, 16
NW = NC * NS
CHUNK = 128

_SC_PARAMS = pltpu.CompilerParams(needs_layout_passes=False)


def _expmap0_scale(ns2):
    ns = jnp.sqrt(ns2)
    un = jnp.maximum(ns, MIN_NORM)
    arg = jnp.clip(SQRT_C * un, -15.0, 15.0)
    g = jnp.tanh(arg) / (SQRT_C * un)
    gn = jnp.maximum(g * ns, MIN_NORM)
    return g * jnp.where(gn > MAXNORM, MAXNORM / gn, 1.0)


def _logmap0_scale(yn2):
    yn = jnp.maximum(jnp.sqrt(yn2), MIN_NORM)
    z = jnp.clip(SQRT_C * yn, -1 + 1e-05, 1 - 1e-05)
    art = 0.5 * jnp.log((1.0 + z) / (1.0 - z))
    return art / (yn * SQRT_C)


def _prep_body(h_ref, ws_ref, rec_ref):
    h = h_ref[...]
    a = jnp.dot(h, ws_ref[...], preferred_element_type=f32)
    ns2 = jnp.sum(h * h, axis=-1, keepdims=True)
    fs = _expmap0_scale(ns2)
    lane = lax.broadcasted_iota(i32, (1, 16), 1)
    rec = a + jnp.where(lane == 8, fs, 0.0) + jnp.where(lane == 9, ns2, 0.0)
    rec_ref[...] = lax.slice(rec, (0, 0), (rec.shape[0], 10))


def _prep_call(h_pad, ws_pad):
    nblk = N_PAD // 512
    return pl.pallas_call(
        _prep_body,
        grid=(nblk,),
        in_specs=[
            pl.BlockSpec((512, IN_DIM), lambda i: (i, 0)),
            pl.BlockSpec((IN_DIM, 16), lambda i: (0, 0)),
        ],
        out_specs=pl.BlockSpec((512, 10), lambda i: (i, 0)),
        out_shape=jax.ShapeDtypeStruct((N_PAD, 10), f32),
    )(h_pad, ws_pad)


def _relprep_body(rp_ref, wr_ref, wqr_ref, b_ref, q_ref, rrec_ref, qrec_ref):
    rp = rp_ref[...]
    a = jnp.dot(rp, wr_ref[...], preferred_element_type=f32)
    nr2 = jnp.sum(rp * rp, axis=-1, keepdims=True)
    gr = _expmap0_scale(nr2)
    lane = lax.broadcasted_iota(i32, (1, 16), 1)
    rec = a + jnp.where(lane == 8, gr, 0.0) + jnp.where(lane == 9, nr2, 0.0)
    rrec_ref[...] = lax.slice(rec, (0, 0), (512, 10))
    col = lax.broadcasted_iota(i32, (512, R_PAD), 1)
    oh = jnp.where(col == q_ref[...], 1.0, 0.0).astype(f32)
    hq = jnp.dot(oh, rp, preferred_element_type=f32)
    qrec = jnp.dot(hq, wqr_ref[...], preferred_element_type=f32) + b_ref[...]
    qrec_ref[...] = lax.slice(qrec, (0, 0), (512, 8))


def _relprep_call(rela_pad, wr_pad, wqr_pad, b_pad, q_rel32):
    return pl.pallas_call(
        _relprep_body,
        in_specs=[
            pl.BlockSpec((R_PAD, IN_DIM), lambda: (0, 0)),
            pl.BlockSpec((IN_DIM, 16), lambda: (0, 0)),
            pl.BlockSpec((IN_DIM, 16), lambda: (0, 0)),
            pl.BlockSpec((1, 16), lambda: (0, 0)),
            pl.BlockSpec((512, 1), lambda: (0, 0)),
        ],
        out_specs=[
            pl.BlockSpec((R_PAD, 10), lambda: (0, 0)),
            pl.BlockSpec((512, 8), lambda: (0, 0)),
        ],
        out_shape=[
            jax.ShapeDtypeStruct((R_PAD, 10), f32),
            jax.ShapeDtypeStruct((512, 8), f32),
        ],
    )(rela_pad, wr_pad, wqr_pad, b_pad, q_rel32)


def _make_gather(e_pad):
    per_tile = e_pad // NW
    nchunk = per_tile // CHUNK
    nrow = e_pad // CHUNK
    mesh = plsc.VectorSubcoreMesh(core_axis_name="c", subcore_axis_name="s")

    @functools.partial(
        pl.kernel,
        out_type=jax.ShapeDtypeStruct((16 * nrow, CHUNK), f32),
        mesh=mesh,
        compiler_params=_SC_PARAMS,
        scratch_types=[
            pltpu.VMEM((N_PAD * 10,), f32),
            pltpu.VMEM((R_PAD * 10,), f32),
            pltpu.VMEM((512 * 8,), f32),
            pltpu.VMEM((CHUNK,), i32),
            pltpu.VMEM((CHUNK,), i32),
            pltpu.VMEM((CHUNK,), i32),
            pltpu.VMEM((16, CHUNK), f32),
            pltpu.SemaphoreType.DMA,
        ],
    )
    def gather(nflat, rflat, qflat, psr, ridx, o2,
               nrec_t, rrec_t, qrec_t, eidx_v, psr_v, ridx_v, fbuf, sem):
        sid = lax.axis_index("s").astype(i32)
        cid = lax.axis_index("c").astype(i32)
        wid = sid * np.int32(NC) + cid
        base = wid * np.int32(per_tile)
        rbase = wid * np.int32(nchunk)

        pltpu.sync_copy(nflat, nrec_t)
        pltpu.sync_copy(rflat, rrec_t)
        pltpu.sync_copy(qflat, qrec_t)

        zv = jnp.zeros((16,), f32)
        for r in range(12, 16):
            for g in range(CHUNK // 16):
                fbuf[r, pl.ds(g * 16, 16)] = zv

        ii = lax.iota(i32, 16)

        @pl.loop(np.int32(0), np.int32(nchunk), step=np.int32(1))
        def _chunk(j):
            jj = j.astype(i32)
            off = base + jj * np.int32(CHUNK)
            for g in range(CHUNK // 16):
                eidx_v[pl.ds(g * 16, 16)] = ii + (off + np.int32(g * 16))
            c1 = pltpu.async_copy(psr.at[eidx_v], psr_v, sem)
            c2 = pltpu.async_copy(ridx.at[eidx_v], ridx_v, sem)
            c1.wait(); c2.wait()
            for g in range(CHUNK // 16):
                sl = pl.ds(g * 16, 16)
                p = psr_v[sl]
                sub = lax.shift_right_logical(p, 9)
                rel = lax.bitwise_and(p, np.int32(511))
                s10 = sub * np.int32(10)
                r10 = rel * np.int32(10)
                q8 = ridx_v[sl] * np.int32(8)
                for k in range(8):
                    v = (plsc.load_gather(nrec_t, [s10 + np.int32(k)])
                         + plsc.load_gather(rrec_t, [r10 + np.int32(k)])
                         + plsc.load_gather(qrec_t, [q8 + np.int32(k)]))
                    fbuf[k, sl] = v
                fbuf[8, sl] = plsc.load_gather(nrec_t, [s10 + np.int32(8)])
                fbuf[9, sl] = plsc.load_gather(nrec_t, [s10 + np.int32(9)])
                fbuf[10, sl] = plsc.load_gather(rrec_t, [r10 + np.int32(8)])
                fbuf[11, sl] = plsc.load_gather(rrec_t, [r10 + np.int32(9)])
            oidx = ii * np.int32(nrow) + (rbase + jj)
            pltpu.sync_copy(fbuf, o2.at[oidx])

    return gather


def _edge_body(wa_ref, x_ref, o_ref):
    x = x_ref[...]
    logit = jnp.zeros(x.shape[1:], f32)
    for k in range(8):
        logit = logit + wa_ref[0, k] * jax.nn.relu(x[k])
    alpha = 1.0 / (1.0 + jnp.exp(-logit))
    fs, ns2, gr, nr2 = x[8], x[9], x[10], x[11]
    x2 = fs * fs * ns2
    y2 = gr * gr * nr2
    t = 1.0 + C * y2
    den = jnp.maximum(1.0 + C * C * x2 * y2, MIN_NORM)
    aa = t / den * fs
    bb = (1.0 - C * x2) / den * gr
    m2 = aa * aa * ns2 + bb * bb * nr2
    pn = jnp.maximum(jnp.sqrt(m2), MIN_NORM)
    pf = jnp.where(pn > MAXNORM, MAXNORM / pn, 1.0)
    scale = _logmap0_scale(pf * pf * m2) * pf * alpha
    o_ref[0] = scale * aa
    o_ref[1] = scale * bb


def _edge_call(wa, o3, e_pad):
    nrow = e_pad // CHUNK
    br = 256
    nblk = nrow // br
    return pl.pallas_call(
        _edge_body,
        grid=(nblk,),
        in_specs=[
            pl.BlockSpec(memory_space=pltpu.SMEM),
            pl.BlockSpec((16, br, CHUNK), lambda i: (0, i, 0)),
        ],
        out_specs=pl.BlockSpec((2, br, CHUNK), lambda i: (0, i, 0)),
        out_shape=jax.ShapeDtypeStruct((2, nrow, CHUNK), f32),
    )(wa, o3)


def _make_scatter(e_pad):
    per_tile = e_pad // NW
    nchunk = per_tile // CHUNK
    rows_per_tile = A_PAD // NS
    mesh = plsc.VectorSubcoreMesh(core_axis_name="c", subcore_axis_name="s")

    @functools.partial(
        pl.kernel,
        out_type=jax.ShapeDtypeStruct((NC, NS, rows_per_tile, IN_DIM), f32),
        mesh=mesh,
        compiler_params=_SC_PARAMS,
        scratch_types=[
            pltpu.VMEM((CHUNK,), i32),
            pltpu.VMEM((CHUNK,), i32),
            pltpu.VMEM((CHUNK,), i32),
            pltpu.VMEM((CHUNK,), i32),
            pltpu.VMEM((CHUNK,), i32),
            pltpu.VMEM((CHUNK,), f32),
            pltpu.VMEM((CHUNK,), f32),
            pltpu.VMEM((CHUNK, IN_DIM), f32),
            pltpu.VMEM((CHUNK, IN_DIM), f32),
            pltpu.VMEM((CHUNK, IN_DIM), f32),
            pltpu.VMEM_SHARED((A_PAD, IN_DIM), f32),
            pltpu.SemaphoreType.DMA,
        ],
    )
    def scatter(hid, rela, psr, obj, w1f, w2f, out,
                eidx_v, psr_v, obj_v, sub_v, rel_v, w1_v, w2_v,
                hs_b, hr_b, msg_b, acc, sem):
        sid = lax.axis_index("s").astype(i32)
        cid = lax.axis_index("c").astype(i32)
        wid = sid * np.int32(NC) + cid
        base = wid * np.int32(per_tile)
        row0 = sid * np.int32(rows_per_tile)
        ii = lax.iota(i32, 16)

        zv = jnp.zeros((16,), f32)
        for e in range(CHUNK):
            for g in range(IN_DIM // 16):
                msg_b[e, pl.ds(g * 16, 16)] = zv

        @pl.loop(np.int32(0), np.int32(4), step=np.int32(1))
        def _zero(k):
            r = row0 + k.astype(i32) * np.int32(CHUNK)
            pltpu.sync_copy(msg_b, acc.at[pl.ds(r, CHUNK)])
        pltpu.sync_copy(msg_b.at[pl.ds(0, rows_per_tile - 4 * CHUNK)],
                        acc.at[pl.ds(row0 + np.int32(4 * CHUNK),
                                     rows_per_tile - 4 * CHUNK)])

        plsc.subcore_barrier()

        @pl.loop(np.int32(0), np.int32(nchunk), step=np.int32(1))
        def _chunk(j):
            off = base + j.astype(i32) * np.int32(CHUNK)
            for g in range(CHUNK // 16):
                eidx_v[pl.ds(g * 16, 16)] = ii + (off + np.int32(g * 16))
            c1 = pltpu.async_copy(psr.at[eidx_v], psr_v, sem)
            c2 = pltpu.async_copy(obj.at[eidx_v], obj_v, sem)
            c3 = pltpu.async_copy(w1f.at[eidx_v], w1_v, sem)
            c4 = pltpu.async_copy(w2f.at[eidx_v], w2_v, sem)
            c1.wait(); c2.wait(); c3.wait(); c4.wait()
            for g in range(CHUNK // 16):
                sl = pl.ds(g * 16, 16)
                p = psr_v[sl]
                sub_v[sl] = lax.shift_right_logical(p, 9)
                rel_v[sl] = lax.bitwise_and(p, np.int32(511))
            g1 = pltpu.async_copy(hid.at[sub_v], hs_b, sem)
            g2 = pltpu.async_copy(rela.at[rel_v], hr_b, sem)
            g1.wait(); g2.wait()

            @pl.loop(np.int32(0), np.int32(CHUNK), step=np.int32(1))
            def _edge(e):
                ee = jnp.full((16,), e, dtype=i32)
                w1v = plsc.load_gather(w1_v, [ee])
                w2v = plsc.load_gather(w2_v, [ee])
                for k in range(IN_DIM // 16):
                    sl = pl.ds(k * 16, 16)
                    msg_b[e, sl] = w1v * hs_b[e, sl] + w2v * hr_b[e, sl]

            pltpu.sync_copy(msg_b, acc.at[obj_v], add=True)

        plsc.subcore_barrier()
        pltpu.sync_copy(acc.at[pl.ds(row0, rows_per_tile)], out.at[cid].at[sid])

    return scatter


def _final_body(p_ref, wh_ref, o_ref):
    g = p_ref[0] + p_ref[1]
    a = lax.dot_general(g, wh_ref[...], (((1,), (1,)), ((), ())),
                        preferred_element_type=f32)
    an2 = jnp.sum(a * a, axis=-1, keepdims=True)
    fa = _expmap0_scale(an2)
    y = fa * a
    o_ref[...] = y * _logmap0_scale(fa * fa * an2)


def _final_call(parts, w_h):
    blk = 632
    nblk = 10112 // blk
    return pl.pallas_call(
        _final_body,
        grid=(nblk,),
        in_specs=[
            pl.BlockSpec((NC, blk, IN_DIM), lambda i: (0, i, 0)),
            pl.BlockSpec((IN_DIM, IN_DIM), lambda i: (0, 0)),
        ],
        out_specs=pl.BlockSpec((blk, IN_DIM), lambda i: (i, 0)),
        out_shape=jax.ShapeDtypeStruct((10112, IN_DIM), f32),
    )(parts, w_h)


def kernel(q_sub, q_rel, hidden, edges, n_node, old_nodes_new_idx, rela_embed,
           Ws_attn_W, Wr_attn_W, Wqr_attn_W, Wqr_attn_b, W_attn_W, W_h_W):
    with jax.enable_x64(False):
        return _run(q_sub, q_rel, hidden, edges, n_node, old_nodes_new_idx,
                    rela_embed, Ws_attn_W, Wr_attn_W, Wqr_attn_W, Wqr_attn_b,
                    W_attn_W, W_h_W)


def _run(q_sub, q_rel, hidden, edges, n_node, old_nodes_new_idx, rela_embed,
         Ws_attn_W, Wr_attn_W, Wqr_attn_W, Wqr_attn_b, W_attn_W, W_h_W):
    hidden = hidden.astype(f32)
    rela = rela_embed.astype(f32)
    n_hid = hidden.shape[0]
    n_rel = rela.shape[0]
    n_edge = edges.shape[0]
    e_pad = ((n_edge + NW * CHUNK * 8 - 1) // (NW * CHUNK * 8)) * (NW * CHUNK * 8)

    h_pad = jnp.pad(hidden, ((0, N_PAD - n_hid), (0, 0)))
    rela_pad = jnp.pad(rela, ((0, R_PAD - n_rel), (0, 0)))
    ws_pad = jnp.pad(Ws_attn_W.astype(f32).T, ((0, 0), (0, 8)))
    wr_pad = jnp.pad(Wr_attn_W.astype(f32).T, ((0, 0), (0, 8)))
    wqr_pad = jnp.pad(Wqr_attn_W.astype(f32).T, ((0, 0), (0, 8)))
    b_pad = jnp.pad(Wqr_attn_b.astype(f32), (0, 8)).reshape(1, 16)
    wa = W_attn_W.astype(f32)
    q_rel32 = q_rel.astype(i32).reshape(512, 1)

    npad = e_pad - n_edge
    sub_c = edges[:, 4].astype(i32)
    rel_c = edges[:, 2].astype(i32)
    psr = jnp.concatenate([sub_c * R_PAD + rel_c, jnp.zeros((npad,), i32)])
    ridx = jnp.concatenate([edges[:, 0].astype(i32), jnp.zeros((npad,), i32)])
    obj = jnp.concatenate([
        (edges[:, 5] + (n_node - n_hid)).astype(i32),
        jnp.full((npad,), N_NODE, i32),
    ])

    nrec = _prep_call(h_pad, ws_pad)
    rrec, qrec = _relprep_call(rela_pad, wr_pad, wqr_pad, b_pad, q_rel32)

    o2 = _make_gather(e_pad)(
        nrec.reshape(-1), rrec.reshape(-1), qrec.reshape(-1), psr, ridx)

    nrow = e_pad // CHUNK
    w2d = _edge_call(wa, o2.reshape(16, nrow, CHUNK), e_pad)

    parts = _make_scatter(e_pad)(
        hidden, rela, psr, obj, w2d[0].reshape(-1), w2d[1].reshape(-1))

    parts = jnp.pad(parts.reshape(NC, A_PAD, IN_DIM),
                    ((0, 0), (0, 10112 - A_PAD), (0, 0)))
    out = _final_call(parts, W_h_W.astype(f32))
    return out[:n_hid]


# speedup vs baseline: 1.1905x; 1.1362x over previous
import functools

import jax
import jax.numpy as jnp
import numpy as np
from jax import lax
from jax.experimental import pallas as pl
from jax.experimental.pallas import tpu as pltpu
from jax.experimental.pallas import tpu_sc as plsc

f32 = jnp.float32
i32 = jnp.int32

MIN_NORM = 1e-15
C = float(np.float32(1e-6))
SQRT_C = float(np.sqrt(np.float32(1e-6)))
MAXNORM = float(np.float32(1.0 - 0.004) / np.float32(SQRT_C))

N_NODE = 10000
N_PAD = 10240
A_PAD = 10112
R_PAD = 512
IN_DIM = 128
NC, NS = 2, 16
NW = NC * NS
CHUNK = 128

_SC_PARAMS = pltpu.CompilerParams(needs_layout_passes=False)


def _expmap0_scale(ns2):
    ns = jnp.sqrt(ns2)
    un = jnp.maximum(ns, MIN_NORM)
    arg = jnp.clip(SQRT_C * un, -15.0, 15.0)
    g = jnp.tanh(arg) / (SQRT_C * un)
    gn = jnp.maximum(g * ns, MIN_NORM)
    return g * jnp.where(gn > MAXNORM, MAXNORM / gn, 1.0)


def _logmap0_scale(yn2):
    yn = jnp.maximum(jnp.sqrt(yn2), MIN_NORM)
    z = jnp.clip(SQRT_C * yn, -1 + 1e-05, 1 - 1e-05)
    art = 0.5 * jnp.log((1.0 + z) / (1.0 - z))
    return art / (yn * SQRT_C)


def _prep_body(h_ref, ws_ref, rec_ref):
    h = h_ref[...]
    a = jnp.dot(h, ws_ref[...], preferred_element_type=f32)
    ns2 = jnp.sum(h * h, axis=-1, keepdims=True)
    fs = _expmap0_scale(ns2)
    lane = lax.broadcasted_iota(i32, (1, 16), 1)
    rec = a + jnp.where(lane == 8, fs, 0.0) + jnp.where(lane == 9, ns2, 0.0)
    rec_ref[...] = lax.slice(rec, (0, 0), (rec.shape[0], 10))


def _prep_call(h_pad, ws_pad):
    nblk = N_PAD // 512
    return pl.pallas_call(
        _prep_body,
        grid=(nblk,),
        in_specs=[
            pl.BlockSpec((512, IN_DIM), lambda i: (i, 0)),
            pl.BlockSpec((IN_DIM, 16), lambda i: (0, 0)),
        ],
        out_specs=pl.BlockSpec((512, 10), lambda i: (i, 0)),
        out_shape=jax.ShapeDtypeStruct((N_PAD, 10), f32),
    )(h_pad, ws_pad)


def _relprep_body(rp_ref, wr_ref, wqr_ref, b_ref, q_ref, rrec_ref, qrec_ref):
    rp = rp_ref[...]
    a = jnp.dot(rp, wr_ref[...], preferred_element_type=f32)
    nr2 = jnp.sum(rp * rp, axis=-1, keepdims=True)
    gr = _expmap0_scale(nr2)
    lane = lax.broadcasted_iota(i32, (1, 16), 1)
    rec = a + jnp.where(lane == 8, gr, 0.0) + jnp.where(lane == 9, nr2, 0.0)
    rrec_ref[...] = lax.slice(rec, (0, 0), (512, 10))
    col = lax.broadcasted_iota(i32, (512, R_PAD), 1)
    oh = jnp.where(col == q_ref[...], 1.0, 0.0).astype(f32)
    hq = jnp.dot(oh, rp, preferred_element_type=f32)
    qrec = jnp.dot(hq, wqr_ref[...], preferred_element_type=f32) + b_ref[...]
    qrec_ref[...] = lax.slice(qrec, (0, 0), (512, 8))


def _relprep_call(rela_pad, wr_pad, wqr_pad, b_pad, q_rel32):
    return pl.pallas_call(
        _relprep_body,
        in_specs=[
            pl.BlockSpec((R_PAD, IN_DIM), lambda: (0, 0)),
            pl.BlockSpec((IN_DIM, 16), lambda: (0, 0)),
            pl.BlockSpec((IN_DIM, 16), lambda: (0, 0)),
            pl.BlockSpec((1, 16), lambda: (0, 0)),
            pl.BlockSpec((512, 1), lambda: (0, 0)),
        ],
        out_specs=[
            pl.BlockSpec((R_PAD, 10), lambda: (0, 0)),
            pl.BlockSpec((512, 8), lambda: (0, 0)),
        ],
        out_shape=[
            jax.ShapeDtypeStruct((R_PAD, 10), f32),
            jax.ShapeDtypeStruct((512, 8), f32),
        ],
    )(rela_pad, wr_pad, wqr_pad, b_pad, q_rel32)


def _make_gather(e_pad):
    per_tile = e_pad // NW
    nchunk = per_tile // CHUNK
    nrow = e_pad // CHUNK
    mesh = plsc.VectorSubcoreMesh(core_axis_name="c", subcore_axis_name="s")

    @functools.partial(
        pl.kernel,
        out_type=jax.ShapeDtypeStruct((16 * nrow, CHUNK), f32),
        mesh=mesh,
        compiler_params=_SC_PARAMS,
        scratch_types=[
            pltpu.VMEM((N_PAD * 10,), f32),
            pltpu.VMEM((R_PAD * 10,), f32),
            pltpu.VMEM((512 * 8,), f32),
            pltpu.VMEM((CHUNK,), i32),
            pltpu.VMEM((CHUNK,), i32),
            pltpu.VMEM((CHUNK,), i32),
            pltpu.VMEM((16, CHUNK), f32),
            pltpu.SemaphoreType.DMA,
        ],
    )
    def gather(nflat, rflat, qflat, psr, ridx, o2,
               nrec_t, rrec_t, qrec_t, eidx_v, psr_v, ridx_v, fbuf, sem):
        sid = lax.axis_index("s").astype(i32)
        cid = lax.axis_index("c").astype(i32)
        wid = sid * np.int32(NC) + cid
        base = wid * np.int32(per_tile)
        rbase = wid * np.int32(nchunk)

        pltpu.sync_copy(nflat, nrec_t)
        pltpu.sync_copy(rflat, rrec_t)
        pltpu.sync_copy(qflat, qrec_t)

        zv = jnp.zeros((16,), f32)
        for r in range(12, 16):
            for g in range(CHUNK // 16):
                fbuf[r, pl.ds(g * 16, 16)] = zv

        ii = lax.iota(i32, 16)

        @pl.loop(np.int32(0), np.int32(nchunk), step=np.int32(1))
        def _chunk(j):
            jj = j.astype(i32)
            off = base + jj * np.int32(CHUNK)
            for g in range(CHUNK // 16):
                eidx_v[pl.ds(g * 16, 16)] = ii + (off + np.int32(g * 16))
            c1 = pltpu.async_copy(psr.at[eidx_v], psr_v, sem)
            c2 = pltpu.async_copy(ridx.at[eidx_v], ridx_v, sem)
            c1.wait(); c2.wait()
            for g in range(CHUNK // 16):
                sl = pl.ds(g * 16, 16)
                p = psr_v[sl]
                sub = lax.shift_right_logical(p, 9)
                rel = lax.bitwise_and(p, np.int32(511))
                s10 = sub * np.int32(10)
                r10 = rel * np.int32(10)
                q8 = ridx_v[sl] * np.int32(8)
                for k in range(8):
                    v = (plsc.load_gather(nrec_t, [s10 + np.int32(k)])
                         + plsc.load_gather(rrec_t, [r10 + np.int32(k)])
                         + plsc.load_gather(qrec_t, [q8 + np.int32(k)]))
                    fbuf[k, sl] = v
                fbuf[8, sl] = plsc.load_gather(nrec_t, [s10 + np.int32(8)])
                fbuf[9, sl] = plsc.load_gather(nrec_t, [s10 + np.int32(9)])
                fbuf[10, sl] = plsc.load_gather(rrec_t, [r10 + np.int32(8)])
                fbuf[11, sl] = plsc.load_gather(rrec_t, [r10 + np.int32(9)])
            oidx = ii * np.int32(nrow) + (rbase + jj)
            pltpu.sync_copy(fbuf, o2.at[oidx])

    return gather


def _edge_body(wa_ref, x_ref, o_ref):
    x = x_ref[...]
    logit = jnp.zeros(x.shape[1:], f32)
    for k in range(8):
        logit = logit + wa_ref[0, k] * jax.nn.relu(x[k])
    alpha = 1.0 / (1.0 + jnp.exp(-logit))
    fs, ns2, gr, nr2 = x[8], x[9], x[10], x[11]
    x2 = fs * fs * ns2
    y2 = gr * gr * nr2
    t = 1.0 + C * y2
    den = jnp.maximum(1.0 + C * C * x2 * y2, MIN_NORM)
    aa = t / den * fs
    bb = (1.0 - C * x2) / den * gr
    m2 = aa * aa * ns2 + bb * bb * nr2
    pn = jnp.maximum(jnp.sqrt(m2), MIN_NORM)
    pf = jnp.where(pn > MAXNORM, MAXNORM / pn, 1.0)
    scale = _logmap0_scale(pf * pf * m2) * pf * alpha
    o_ref[0] = scale * aa
    o_ref[1] = scale * bb


def _edge_call(wa, o3, e_pad):
    nrow = e_pad // CHUNK
    br = 256
    nblk = nrow // br
    return pl.pallas_call(
        _edge_body,
        grid=(nblk,),
        in_specs=[
            pl.BlockSpec(memory_space=pltpu.SMEM),
            pl.BlockSpec((16, br, CHUNK), lambda i: (0, i, 0)),
        ],
        out_specs=pl.BlockSpec((2, br, CHUNK), lambda i: (0, i, 0)),
        out_shape=jax.ShapeDtypeStruct((2, nrow, CHUNK), f32),
    )(wa, o3)


def _make_scatter(e_pad):
    per_tile = e_pad // NW
    nchunk = per_tile // CHUNK
    rows_per_tile = A_PAD // NS
    mesh = plsc.VectorSubcoreMesh(core_axis_name="c", subcore_axis_name="s")

    @functools.partial(
        pl.kernel,
        out_type=jax.ShapeDtypeStruct((NC, NS, rows_per_tile, IN_DIM), f32),
        mesh=mesh,
        compiler_params=_SC_PARAMS,
        scratch_types=[
            pltpu.VMEM((CHUNK,), i32),
            pltpu.VMEM((CHUNK,), i32),
            pltpu.VMEM((CHUNK,), i32),
            pltpu.VMEM((CHUNK,), i32),
            pltpu.VMEM((CHUNK,), i32),
            pltpu.VMEM((CHUNK,), f32),
            pltpu.VMEM((CHUNK,), f32),
            pltpu.VMEM((CHUNK, IN_DIM), f32),
            pltpu.VMEM((CHUNK, IN_DIM), f32),
            pltpu.VMEM((CHUNK, IN_DIM), f32),
            pltpu.VMEM_SHARED((A_PAD, IN_DIM), f32),
            pltpu.SemaphoreType.DMA,
        ],
    )
    def scatter(hid, rela, psr, obj, w1f, w2f, out,
                eidx_v, psr_v, obj_v, sub_v, rel_v, w1_v, w2_v,
                hs_b, hr_b, msg_b, acc, sem):
        sid = lax.axis_index("s").astype(i32)
        cid = lax.axis_index("c").astype(i32)
        wid = sid * np.int32(NC) + cid
        base = wid * np.int32(per_tile)
        row0 = sid * np.int32(rows_per_tile)
        ii = lax.iota(i32, 16)

        zv = jnp.zeros((16,), f32)
        for e in range(CHUNK):
            for g in range(IN_DIM // 16):
                msg_b[e, pl.ds(g * 16, 16)] = zv

        @pl.loop(np.int32(0), np.int32(4), step=np.int32(1))
        def _zero(k):
            r = row0 + k.astype(i32) * np.int32(CHUNK)
            pltpu.sync_copy(msg_b, acc.at[pl.ds(r, CHUNK)])
        pltpu.sync_copy(msg_b.at[pl.ds(0, rows_per_tile - 4 * CHUNK)],
                        acc.at[pl.ds(row0 + np.int32(4 * CHUNK),
                                     rows_per_tile - 4 * CHUNK)])

        plsc.subcore_barrier()

        @pl.loop(np.int32(0), np.int32(nchunk), step=np.int32(1))
        def _chunk(j):
            off = base + j.astype(i32) * np.int32(CHUNK)
            for g in range(CHUNK // 16):
                eidx_v[pl.ds(g * 16, 16)] = ii + (off + np.int32(g * 16))
            c1 = pltpu.async_copy(psr.at[eidx_v], psr_v, sem)
            c2 = pltpu.async_copy(obj.at[eidx_v], obj_v, sem)
            c3 = pltpu.async_copy(w1f.at[eidx_v], w1_v, sem)
            c4 = pltpu.async_copy(w2f.at[eidx_v], w2_v, sem)
            c1.wait(); c2.wait(); c3.wait(); c4.wait()
            for g in range(CHUNK // 16):
                sl = pl.ds(g * 16, 16)
                p = psr_v[sl]
                sub_v[sl] = lax.shift_right_logical(p, 9)
                rel_v[sl] = lax.bitwise_and(p, np.int32(511))
            h = CHUNK // 2
            g1 = pltpu.async_copy(hid.at[sub_v.at[pl.ds(0, h)]],
                                  hs_b.at[pl.ds(0, h)], sem)
            g2 = pltpu.async_copy(rela.at[rel_v.at[pl.ds(0, h)]],
                                  hr_b.at[pl.ds(0, h)], sem)
            g3 = pltpu.async_copy(hid.at[sub_v.at[pl.ds(h, h)]],
                                  hs_b.at[pl.ds(h, h)], sem)
            g4 = pltpu.async_copy(rela.at[rel_v.at[pl.ds(h, h)]],
                                  hr_b.at[pl.ds(h, h)], sem)
            g1.wait(); g2.wait()

            @pl.loop(np.int32(0), np.int32(h), step=np.int32(1))
            def _edge_lo(e):
                ee = jnp.full((16,), e, dtype=i32)
                w1v = plsc.load_gather(w1_v, [ee])
                w2v = plsc.load_gather(w2_v, [ee])
                for k in range(IN_DIM // 16):
                    sl = pl.ds(k * 16, 16)
                    msg_b[e, sl] = w1v * hs_b[e, sl] + w2v * hr_b[e, sl]

            g3.wait(); g4.wait()

            @pl.loop(np.int32(h), np.int32(CHUNK), step=np.int32(1))
            def _edge_hi(e):
                ee = jnp.full((16,), e, dtype=i32)
                w1v = plsc.load_gather(w1_v, [ee])
                w2v = plsc.load_gather(w2_v, [ee])
                for k in range(IN_DIM // 16):
                    sl = pl.ds(k * 16, 16)
                    msg_b[e, sl] = w1v * hs_b[e, sl] + w2v * hr_b[e, sl]

            pltpu.sync_copy(msg_b, acc.at[obj_v], add=True)

        plsc.subcore_barrier()
        pltpu.sync_copy(acc.at[pl.ds(row0, rows_per_tile)], out.at[cid].at[sid])

    return scatter


def _final_body(p_ref, wh_ref, o_ref):
    g = p_ref[0] + p_ref[1]
    a = lax.dot_general(g, wh_ref[...], (((1,), (1,)), ((), ())),
                        preferred_element_type=f32)
    an2 = jnp.sum(a * a, axis=-1, keepdims=True)
    fa = _expmap0_scale(an2)
    y = fa * a
    o_ref[...] = y * _logmap0_scale(fa * fa * an2)


def _final_call(parts, w_h):
    blk = 632
    nblk = 10112 // blk
    return pl.pallas_call(
        _final_body,
        grid=(nblk,),
        in_specs=[
            pl.BlockSpec((NC, blk, IN_DIM), lambda i: (0, i, 0)),
            pl.BlockSpec((IN_DIM, IN_DIM), lambda i: (0, 0)),
        ],
        out_specs=pl.BlockSpec((blk, IN_DIM), lambda i: (i, 0)),
        out_shape=jax.ShapeDtypeStruct((10112, IN_DIM), f32),
    )(parts, w_h)


def kernel(q_sub, q_rel, hidden, edges, n_node, old_nodes_new_idx, rela_embed,
           Ws_attn_W, Wr_attn_W, Wqr_attn_W, Wqr_attn_b, W_attn_W, W_h_W):
    with jax.enable_x64(False):
        return _run(q_sub, q_rel, hidden, edges, n_node, old_nodes_new_idx,
                    rela_embed, Ws_attn_W, Wr_attn_W, Wqr_attn_W, Wqr_attn_b,
                    W_attn_W, W_h_W)


def _run(q_sub, q_rel, hidden, edges, n_node, old_nodes_new_idx, rela_embed,
         Ws_attn_W, Wr_attn_W, Wqr_attn_W, Wqr_attn_b, W_attn_W, W_h_W):
    hidden = hidden.astype(f32)
    rela = rela_embed.astype(f32)
    n_hid = hidden.shape[0]
    n_rel = rela.shape[0]
    n_edge = edges.shape[0]
    e_pad = ((n_edge + NW * CHUNK * 8 - 1) // (NW * CHUNK * 8)) * (NW * CHUNK * 8)

    h_pad = jnp.pad(hidden, ((0, N_PAD - n_hid), (0, 0)))
    rela_pad = jnp.pad(rela, ((0, R_PAD - n_rel), (0, 0)))
    ws_pad = jnp.pad(Ws_attn_W.astype(f32).T, ((0, 0), (0, 8)))
    wr_pad = jnp.pad(Wr_attn_W.astype(f32).T, ((0, 0), (0, 8)))
    wqr_pad = jnp.pad(Wqr_attn_W.astype(f32).T, ((0, 0), (0, 8)))
    b_pad = jnp.pad(Wqr_attn_b.astype(f32), (0, 8)).reshape(1, 16)
    wa = W_attn_W.astype(f32)
    q_rel32 = q_rel.astype(i32).reshape(512, 1)

    npad = e_pad - n_edge
    sub_c = edges[:, 4].astype(i32)
    rel_c = edges[:, 2].astype(i32)
    psr = jnp.concatenate([sub_c * R_PAD + rel_c, jnp.zeros((npad,), i32)])
    ridx = jnp.concatenate([edges[:, 0].astype(i32), jnp.zeros((npad,), i32)])
    obj = jnp.concatenate([
        (edges[:, 5] + (n_node - n_hid)).astype(i32),
        jnp.full((npad,), N_NODE, i32),
    ])

    nrec = _prep_call(h_pad, ws_pad)
    rrec, qrec = _relprep_call(rela_pad, wr_pad, wqr_pad, b_pad, q_rel32)

    o2 = _make_gather(e_pad)(
        nrec.reshape(-1), rrec.reshape(-1), qrec.reshape(-1), psr, ridx)

    nrow = e_pad // CHUNK
    w2d = _edge_call(wa, o2.reshape(16, nrow, CHUNK), e_pad)

    parts = _make_scatter(e_pad)(
        hidden, rela, psr, obj, w2d[0].reshape(-1), w2d[1].reshape(-1))

    parts = jnp.pad(parts.reshape(NC, A_PAD, IN_DIM),
                    ((0, 0), (0, 10112 - A_PAD), (0, 0)))
    out = _final_call(parts, W_h_W.astype(f32))
    return out[:n_hid]


# speedup vs baseline: 1.2821x; 1.0769x over previous
import functools

import jax
import jax.numpy as jnp
import numpy as np
from jax import lax
from jax.experimental import pallas as pl
from jax.experimental.pallas import tpu as pltpu
from jax.experimental.pallas import tpu_sc as plsc

f32 = jnp.float32
i32 = jnp.int32

MIN_NORM = 1e-15
C = float(np.float32(1e-6))
SQRT_C = float(np.sqrt(np.float32(1e-6)))
MAXNORM = float(np.float32(1.0 - 0.004) / np.float32(SQRT_C))

N_NODE = 10000
N_PAD = 10240
A_PAD = 10112
R_PAD = 512
IN_DIM = 128
NC, NS = 2, 16
NW = NC * NS
CHUNK = 128

_SC_PARAMS = pltpu.CompilerParams(needs_layout_passes=False)


def _expmap0_scale(ns2):
    ns = jnp.sqrt(ns2)
    un = jnp.maximum(ns, MIN_NORM)
    arg = jnp.clip(SQRT_C * un, -15.0, 15.0)
    g = jnp.tanh(arg) / (SQRT_C * un)
    gn = jnp.maximum(g * ns, MIN_NORM)
    return g * jnp.where(gn > MAXNORM, MAXNORM / gn, 1.0)


def _logmap0_scale(yn2):
    yn = jnp.maximum(jnp.sqrt(yn2), MIN_NORM)
    z = jnp.clip(SQRT_C * yn, -1 + 1e-05, 1 - 1e-05)
    art = 0.5 * jnp.log((1.0 + z) / (1.0 - z))
    return art / (yn * SQRT_C)


def _prep_body(h_ref, ws_ref, rec_ref):
    h = h_ref[...]
    a = jnp.dot(h, ws_ref[...], preferred_element_type=f32)
    ns2 = jnp.sum(h * h, axis=-1, keepdims=True)
    fs = _expmap0_scale(ns2)
    lane = lax.broadcasted_iota(i32, (1, 16), 1)
    rec = a + jnp.where(lane == 8, fs, 0.0) + jnp.where(lane == 9, ns2, 0.0)
    rec_ref[...] = lax.slice(rec, (0, 0), (rec.shape[0], 10))


def _prep_call(h_pad, ws_pad):
    nblk = N_PAD // 512
    return pl.pallas_call(
        _prep_body,
        grid=(nblk,),
        in_specs=[
            pl.BlockSpec((512, IN_DIM), lambda i: (i, 0)),
            pl.BlockSpec((IN_DIM, 16), lambda i: (0, 0)),
        ],
        out_specs=pl.BlockSpec((512, 10), lambda i: (i, 0)),
        out_shape=jax.ShapeDtypeStruct((N_PAD, 10), f32),
    )(h_pad, ws_pad)


def _relprep_body(rp_ref, wr_ref, wqr_ref, b_ref, q_ref, rrec_ref, qrec_ref):
    rp = rp_ref[...]
    a = jnp.dot(rp, wr_ref[...], preferred_element_type=f32)
    nr2 = jnp.sum(rp * rp, axis=-1, keepdims=True)
    gr = _expmap0_scale(nr2)
    lane = lax.broadcasted_iota(i32, (1, 16), 1)
    rec = a + jnp.where(lane == 8, gr, 0.0) + jnp.where(lane == 9, nr2, 0.0)
    rrec_ref[...] = lax.slice(rec, (0, 0), (512, 10))
    col = lax.broadcasted_iota(i32, (512, R_PAD), 1)
    oh = jnp.where(col == q_ref[...], 1.0, 0.0).astype(f32)
    hq = jnp.dot(oh, rp, preferred_element_type=f32)
    qrec = jnp.dot(hq, wqr_ref[...], preferred_element_type=f32) + b_ref[...]
    qrec_ref[...] = lax.slice(qrec, (0, 0), (512, 8))


def _relprep_call(rela_pad, wr_pad, wqr_pad, b_pad, q_rel32):
    return pl.pallas_call(
        _relprep_body,
        in_specs=[
            pl.BlockSpec((R_PAD, IN_DIM), lambda: (0, 0)),
            pl.BlockSpec((IN_DIM, 16), lambda: (0, 0)),
            pl.BlockSpec((IN_DIM, 16), lambda: (0, 0)),
            pl.BlockSpec((1, 16), lambda: (0, 0)),
            pl.BlockSpec((512, 1), lambda: (0, 0)),
        ],
        out_specs=[
            pl.BlockSpec((R_PAD, 10), lambda: (0, 0)),
            pl.BlockSpec((512, 8), lambda: (0, 0)),
        ],
        out_shape=[
            jax.ShapeDtypeStruct((R_PAD, 10), f32),
            jax.ShapeDtypeStruct((512, 8), f32),
        ],
    )(rela_pad, wr_pad, wqr_pad, b_pad, q_rel32)


def _make_gather(e_pad):
    per_tile = e_pad // NW
    nchunk = per_tile // CHUNK
    nrow = e_pad // CHUNK
    mesh = plsc.VectorSubcoreMesh(core_axis_name="c", subcore_axis_name="s")

    @functools.partial(
        pl.kernel,
        out_type=jax.ShapeDtypeStruct((16 * nrow, CHUNK), f32),
        mesh=mesh,
        compiler_params=_SC_PARAMS,
        scratch_types=[
            pltpu.VMEM((N_PAD * 10,), f32),
            pltpu.VMEM((R_PAD * 10,), f32),
            pltpu.VMEM((512 * 8,), f32),
            pltpu.VMEM((CHUNK,), i32),
            pltpu.VMEM((CHUNK,), i32),
            pltpu.VMEM((CHUNK,), i32),
            pltpu.VMEM((16, CHUNK), f32),
            pltpu.SemaphoreType.DMA,
        ],
    )
    def gather(nflat, rflat, qflat, psr, ridx, o2,
               nrec_t, rrec_t, qrec_t, eidx_v, psr_v, ridx_v, fbuf, sem):
        sid = lax.axis_index("s").astype(i32)
        cid = lax.axis_index("c").astype(i32)
        wid = sid * np.int32(NC) + cid
        base = wid * np.int32(per_tile)
        rbase = wid * np.int32(nchunk)

        pltpu.sync_copy(nflat, nrec_t)
        pltpu.sync_copy(rflat, rrec_t)
        pltpu.sync_copy(qflat, qrec_t)

        zv = jnp.zeros((16,), f32)
        for r in range(12, 16):
            for g in range(CHUNK // 16):
                fbuf[r, pl.ds(g * 16, 16)] = zv

        ii = lax.iota(i32, 16)

        @pl.loop(np.int32(0), np.int32(nchunk), step=np.int32(1))
        def _chunk(j):
            jj = j.astype(i32)
            off = base + jj * np.int32(CHUNK)
            for g in range(CHUNK // 16):
                eidx_v[pl.ds(g * 16, 16)] = ii + (off + np.int32(g * 16))
            c1 = pltpu.async_copy(psr.at[eidx_v], psr_v, sem)
            c2 = pltpu.async_copy(ridx.at[eidx_v], ridx_v, sem)
            c1.wait(); c2.wait()
            for g in range(CHUNK // 16):
                sl = pl.ds(g * 16, 16)
                p = psr_v[sl]
                sub = lax.shift_right_logical(p, 9)
                rel = lax.bitwise_and(p, np.int32(511))
                s10 = sub * np.int32(10)
                r10 = rel * np.int32(10)
                q8 = ridx_v[sl] * np.int32(8)
                for k in range(8):
                    v = (plsc.load_gather(nrec_t, [s10 + np.int32(k)])
                         + plsc.load_gather(rrec_t, [r10 + np.int32(k)])
                         + plsc.load_gather(qrec_t, [q8 + np.int32(k)]))
                    fbuf[k, sl] = v
                fbuf[8, sl] = plsc.load_gather(nrec_t, [s10 + np.int32(8)])
                fbuf[9, sl] = plsc.load_gather(nrec_t, [s10 + np.int32(9)])
                fbuf[10, sl] = plsc.load_gather(rrec_t, [r10 + np.int32(8)])
                fbuf[11, sl] = plsc.load_gather(rrec_t, [r10 + np.int32(9)])
            oidx = ii * np.int32(nrow) + (rbase + jj)
            pltpu.sync_copy(fbuf, o2.at[oidx])

    return gather


def _edge_body(wa_ref, x_ref, o_ref):
    x = x_ref[...]
    logit = jnp.zeros(x.shape[1:], f32)
    for k in range(8):
        logit = logit + wa_ref[0, k] * jax.nn.relu(x[k])
    alpha = 1.0 / (1.0 + jnp.exp(-logit))
    fs, ns2, gr, nr2 = x[8], x[9], x[10], x[11]
    x2 = fs * fs * ns2
    y2 = gr * gr * nr2
    t = 1.0 + C * y2
    den = jnp.maximum(1.0 + C * C * x2 * y2, MIN_NORM)
    aa = t / den * fs
    bb = (1.0 - C * x2) / den * gr
    m2 = aa * aa * ns2 + bb * bb * nr2
    pn = jnp.maximum(jnp.sqrt(m2), MIN_NORM)
    pf = jnp.where(pn > MAXNORM, MAXNORM / pn, 1.0)
    scale = _logmap0_scale(pf * pf * m2) * pf * alpha
    o_ref[0] = scale * aa
    o_ref[1] = scale * bb


def _edge_call(wa, o3, e_pad):
    nrow = e_pad // CHUNK
    br = 256
    nblk = nrow // br
    return pl.pallas_call(
        _edge_body,
        grid=(nblk,),
        in_specs=[
            pl.BlockSpec(memory_space=pltpu.SMEM),
            pl.BlockSpec((16, br, CHUNK), lambda i: (0, i, 0)),
        ],
        out_specs=pl.BlockSpec((2, br, CHUNK), lambda i: (0, i, 0)),
        out_shape=jax.ShapeDtypeStruct((2, nrow, CHUNK), f32),
    )(wa, o3)


def _make_scatter(e_pad):
    per_tile = e_pad // NW
    nchunk = per_tile // CHUNK
    rows_per_tile = A_PAD // NS
    mesh = plsc.VectorSubcoreMesh(core_axis_name="c", subcore_axis_name="s")

    @functools.partial(
        pl.kernel,
        out_type=jax.ShapeDtypeStruct((NC, NS, rows_per_tile, IN_DIM), f32),
        mesh=mesh,
        compiler_params=_SC_PARAMS,
        scratch_types=[
            pltpu.VMEM((CHUNK,), i32),
            pltpu.VMEM((CHUNK,), i32),
            pltpu.VMEM((CHUNK,), i32),
            pltpu.VMEM((CHUNK,), i32),
            pltpu.VMEM((CHUNK,), i32),
            pltpu.VMEM((CHUNK,), f32),
            pltpu.VMEM((CHUNK,), f32),
            pltpu.VMEM((CHUNK, IN_DIM), f32),
            pltpu.VMEM((CHUNK, IN_DIM), f32),
            pltpu.VMEM((CHUNK, IN_DIM), f32),
            pltpu.VMEM_SHARED((A_PAD, IN_DIM), f32),
            pltpu.SemaphoreType.DMA,
        ],
    )
    def scatter(hid, rela, psr, obj, w1f, w2f, out,
                eidx_v, psr_v, obj_v, sub_v, rel_v, w1_v, w2_v,
                hs_b, hr_b, msg_b, acc, sem):
        sid = lax.axis_index("s").astype(i32)
        cid = lax.axis_index("c").astype(i32)
        wid = sid * np.int32(NC) + cid
        base = wid * np.int32(per_tile)
        row0 = sid * np.int32(rows_per_tile)
        ii = lax.iota(i32, 16)

        zv = jnp.zeros((16,), f32)
        for e in range(CHUNK):
            for g in range(IN_DIM // 16):
                msg_b[e, pl.ds(g * 16, 16)] = zv

        @pl.loop(np.int32(0), np.int32(4), step=np.int32(1))
        def _zero(k):
            r = row0 + k.astype(i32) * np.int32(CHUNK)
            pltpu.sync_copy(msg_b, acc.at[pl.ds(r, CHUNK)])
        pltpu.sync_copy(msg_b.at[pl.ds(0, rows_per_tile - 4 * CHUNK)],
                        acc.at[pl.ds(row0 + np.int32(4 * CHUNK),
                                     rows_per_tile - 4 * CHUNK)])

        plsc.subcore_barrier()

        @pl.loop(np.int32(0), np.int32(nchunk), step=np.int32(1))
        def _chunk(j):
            off = base + j.astype(i32) * np.int32(CHUNK)
            for g in range(CHUNK // 16):
                eidx_v[pl.ds(g * 16, 16)] = ii + (off + np.int32(g * 16))
            c1 = pltpu.async_copy(psr.at[eidx_v], psr_v, sem)
            c2 = pltpu.async_copy(obj.at[eidx_v], obj_v, sem)
            c3 = pltpu.async_copy(w1f.at[eidx_v], w1_v, sem)
            c4 = pltpu.async_copy(w2f.at[eidx_v], w2_v, sem)
            c1.wait(); c2.wait(); c3.wait(); c4.wait()
            for g in range(CHUNK // 16):
                sl = pl.ds(g * 16, 16)
                p = psr_v[sl]
                sub_v[sl] = lax.shift_right_logical(p, 9)
                rel_v[sl] = lax.bitwise_and(p, np.int32(511))
            q = CHUNK // 4
            hs_g = []
            for piece in range(4):
                o = piece * q
                hs_g.append(pltpu.async_copy(
                    hid.at[sub_v.at[pl.ds(o, q)]], hs_b.at[pl.ds(o, q)], sem))
                hs_g.append(pltpu.async_copy(
                    rela.at[rel_v.at[pl.ds(o, q)]], hr_b.at[pl.ds(o, q)], sem))

            for piece in range(4):
                hs_g[2 * piece].wait()
                hs_g[2 * piece + 1].wait()

                @pl.loop(np.int32(piece * q), np.int32((piece + 1) * q),
                         step=np.int32(1))
                def _edge(e):
                    ee = jnp.full((16,), e, dtype=i32)
                    w1v = plsc.load_gather(w1_v, [ee])
                    w2v = plsc.load_gather(w2_v, [ee])
                    for k in range(IN_DIM // 16):
                        sl = pl.ds(k * 16, 16)
                        msg_b[e, sl] = w1v * hs_b[e, sl] + w2v * hr_b[e, sl]

            pltpu.sync_copy(msg_b, acc.at[obj_v], add=True)

        plsc.subcore_barrier()
        pltpu.sync_copy(acc.at[pl.ds(row0, rows_per_tile)], out.at[cid].at[sid])

    return scatter


def _final_body(p_ref, wh_ref, o_ref):
    g = p_ref[0] + p_ref[1]
    a = lax.dot_general(g, wh_ref[...], (((1,), (1,)), ((), ())),
                        preferred_element_type=f32)
    an2 = jnp.sum(a * a, axis=-1, keepdims=True)
    fa = _expmap0_scale(an2)
    y = fa * a
    o_ref[...] = y * _logmap0_scale(fa * fa * an2)


def _final_call(parts, w_h):
    blk = 632
    nblk = 10112 // blk
    return pl.pallas_call(
        _final_body,
        grid=(nblk,),
        in_specs=[
            pl.BlockSpec((NC, blk, IN_DIM), lambda i: (0, i, 0)),
            pl.BlockSpec((IN_DIM, IN_DIM), lambda i: (0, 0)),
        ],
        out_specs=pl.BlockSpec((blk, IN_DIM), lambda i: (i, 0)),
        out_shape=jax.ShapeDtypeStruct((10112, IN_DIM), f32),
    )(parts, w_h)


def kernel(q_sub, q_rel, hidden, edges, n_node, old_nodes_new_idx, rela_embed,
           Ws_attn_W, Wr_attn_W, Wqr_attn_W, Wqr_attn_b, W_attn_W, W_h_W):
    with jax.enable_x64(False):
        return _run(q_sub, q_rel, hidden, edges, n_node, old_nodes_new_idx,
                    rela_embed, Ws_attn_W, Wr_attn_W, Wqr_attn_W, Wqr_attn_b,
                    W_attn_W, W_h_W)


def _run(q_sub, q_rel, hidden, edges, n_node, old_nodes_new_idx, rela_embed,
         Ws_attn_W, Wr_attn_W, Wqr_attn_W, Wqr_attn_b, W_attn_W, W_h_W):
    hidden = hidden.astype(f32)
    rela = rela_embed.astype(f32)
    n_hid = hidden.shape[0]
    n_rel = rela.shape[0]
    n_edge = edges.shape[0]
    e_pad = ((n_edge + NW * CHUNK * 8 - 1) // (NW * CHUNK * 8)) * (NW * CHUNK * 8)

    h_pad = jnp.pad(hidden, ((0, N_PAD - n_hid), (0, 0)))
    rela_pad = jnp.pad(rela, ((0, R_PAD - n_rel), (0, 0)))
    ws_pad = jnp.pad(Ws_attn_W.astype(f32).T, ((0, 0), (0, 8)))
    wr_pad = jnp.pad(Wr_attn_W.astype(f32).T, ((0, 0), (0, 8)))
    wqr_pad = jnp.pad(Wqr_attn_W.astype(f32).T, ((0, 0), (0, 8)))
    b_pad = jnp.pad(Wqr_attn_b.astype(f32), (0, 8)).reshape(1, 16)
    wa = W_attn_W.astype(f32)
    q_rel32 = q_rel.astype(i32).reshape(512, 1)

    npad = e_pad - n_edge
    sub_c = edges[:, 4].astype(i32)
    rel_c = edges[:, 2].astype(i32)
    psr = jnp.concatenate([sub_c * R_PAD + rel_c, jnp.zeros((npad,), i32)])
    ridx = jnp.concatenate([edges[:, 0].astype(i32), jnp.zeros((npad,), i32)])
    obj = jnp.concatenate([
        (edges[:, 5] + (n_node - n_hid)).astype(i32),
        jnp.full((npad,), N_NODE, i32),
    ])

    nrec = _prep_call(h_pad, ws_pad)
    rrec, qrec = _relprep_call(rela_pad, wr_pad, wqr_pad, b_pad, q_rel32)

    o2 = _make_gather(e_pad)(
        nrec.reshape(-1), rrec.reshape(-1), qrec.reshape(-1), psr, ridx)

    nrow = e_pad // CHUNK
    w2d = _edge_call(wa, o2.reshape(16, nrow, CHUNK), e_pad)

    parts = _make_scatter(e_pad)(
        hidden, rela, psr, obj, w2d[0].reshape(-1), w2d[1].reshape(-1))

    parts = jnp.pad(parts.reshape(NC, A_PAD, IN_DIM),
                    ((0, 0), (0, 10112 - A_PAD), (0, 0)))
    out = _final_call(parts, W_h_W.astype(f32))
    return out[:n_hid]


# speedup vs baseline: 1.3356x; 1.0417x over previous
import functools

import jax
import jax.numpy as jnp
import numpy as np
from jax import lax
from jax.experimental import pallas as pl
from jax.experimental.pallas import tpu as pltpu
from jax.experimental.pallas import tpu_sc as plsc

f32 = jnp.float32
i32 = jnp.int32

MIN_NORM = 1e-15
C = float(np.float32(1e-6))
SQRT_C = float(np.sqrt(np.float32(1e-6)))
MAXNORM = float(np.float32(1.0 - 0.004) / np.float32(SQRT_C))

N_NODE = 10000
N_PAD = 10240
A_PAD = 10112
R_PAD = 512
IN_DIM = 128
NC, NS = 2, 16
NW = NC * NS
CHUNK = 128

_SC_PARAMS = pltpu.CompilerParams(needs_layout_passes=False)


def _expmap0_scale(ns2):
    ns = jnp.sqrt(ns2)
    un = jnp.maximum(ns, MIN_NORM)
    arg = jnp.clip(SQRT_C * un, -15.0, 15.0)
    g = jnp.tanh(arg) / (SQRT_C * un)
    gn = jnp.maximum(g * ns, MIN_NORM)
    return g * jnp.where(gn > MAXNORM, MAXNORM / gn, 1.0)


def _logmap0_scale(yn2):
    yn = jnp.maximum(jnp.sqrt(yn2), MIN_NORM)
    z = jnp.clip(SQRT_C * yn, -1 + 1e-05, 1 - 1e-05)
    art = 0.5 * jnp.log((1.0 + z) / (1.0 - z))
    return art / (yn * SQRT_C)


def _prep_body(h_ref, ws_ref, rec_ref):
    h = h_ref[...]
    a = jnp.dot(h, ws_ref[...], preferred_element_type=f32)
    ns2 = jnp.sum(h * h, axis=-1, keepdims=True)
    fs = _expmap0_scale(ns2)
    lane = lax.broadcasted_iota(i32, (1, 16), 1)
    rec = a + jnp.where(lane == 8, fs, 0.0) + jnp.where(lane == 9, ns2, 0.0)
    rec_ref[...] = lax.slice(rec, (0, 0), (rec.shape[0], 10))


def _prep_call(h_pad, ws_pad):
    nblk = N_PAD // 512
    return pl.pallas_call(
        _prep_body,
        grid=(nblk,),
        in_specs=[
            pl.BlockSpec((512, IN_DIM), lambda i: (i, 0)),
            pl.BlockSpec((IN_DIM, 16), lambda i: (0, 0)),
        ],
        out_specs=pl.BlockSpec((512, 10), lambda i: (i, 0)),
        out_shape=jax.ShapeDtypeStruct((N_PAD, 10), f32),
    )(h_pad, ws_pad)


def _relprep_body(rp_ref, wr_ref, wqr_ref, b_ref, q_ref, rrec_ref, qrec_ref):
    rp = rp_ref[...]
    a = jnp.dot(rp, wr_ref[...], preferred_element_type=f32)
    nr2 = jnp.sum(rp * rp, axis=-1, keepdims=True)
    gr = _expmap0_scale(nr2)
    lane = lax.broadcasted_iota(i32, (1, 16), 1)
    rec = a + jnp.where(lane == 8, gr, 0.0) + jnp.where(lane == 9, nr2, 0.0)
    rrec_ref[...] = lax.slice(rec, (0, 0), (512, 10))
    col = lax.broadcasted_iota(i32, (512, R_PAD), 1)
    oh = jnp.where(col == q_ref[...], 1.0, 0.0).astype(f32)
    hq = jnp.dot(oh, rp, preferred_element_type=f32)
    qrec = jnp.dot(hq, wqr_ref[...], preferred_element_type=f32) + b_ref[...]
    qrec_ref[...] = lax.slice(qrec, (0, 0), (512, 8))


def _relprep_call(rela_pad, wr_pad, wqr_pad, b_pad, q_rel32):
    return pl.pallas_call(
        _relprep_body,
        in_specs=[
            pl.BlockSpec((R_PAD, IN_DIM), lambda: (0, 0)),
            pl.BlockSpec((IN_DIM, 16), lambda: (0, 0)),
            pl.BlockSpec((IN_DIM, 16), lambda: (0, 0)),
            pl.BlockSpec((1, 16), lambda: (0, 0)),
            pl.BlockSpec((512, 1), lambda: (0, 0)),
        ],
        out_specs=[
            pl.BlockSpec((R_PAD, 10), lambda: (0, 0)),
            pl.BlockSpec((512, 8), lambda: (0, 0)),
        ],
        out_shape=[
            jax.ShapeDtypeStruct((R_PAD, 10), f32),
            jax.ShapeDtypeStruct((512, 8), f32),
        ],
    )(rela_pad, wr_pad, wqr_pad, b_pad, q_rel32)


def _make_gather(e_pad):
    per_tile = e_pad // NW
    nchunk = per_tile // CHUNK
    nrow = e_pad // CHUNK
    mesh = plsc.VectorSubcoreMesh(core_axis_name="c", subcore_axis_name="s")

    @functools.partial(
        pl.kernel,
        out_type=jax.ShapeDtypeStruct((16 * nrow, CHUNK), f32),
        mesh=mesh,
        compiler_params=_SC_PARAMS,
        scratch_types=[
            pltpu.VMEM((N_PAD * 10,), f32),
            pltpu.VMEM((R_PAD * 10,), f32),
            pltpu.VMEM((512 * 8,), f32),
            pltpu.VMEM((CHUNK,), i32),
            pltpu.VMEM((CHUNK,), i32),
            pltpu.VMEM((CHUNK,), i32),
            pltpu.VMEM((16, CHUNK), f32),
            pltpu.SemaphoreType.DMA,
        ],
    )
    def gather(nflat, rflat, qflat, psr, ridx, o2,
               nrec_t, rrec_t, qrec_t, eidx_v, psr_v, ridx_v, fbuf, sem):
        sid = lax.axis_index("s").astype(i32)
        cid = lax.axis_index("c").astype(i32)
        wid = sid * np.int32(NC) + cid
        base = wid * np.int32(per_tile)
        rbase = wid * np.int32(nchunk)

        pltpu.sync_copy(nflat, nrec_t)
        pltpu.sync_copy(rflat, rrec_t)
        pltpu.sync_copy(qflat, qrec_t)

        zv = jnp.zeros((16,), f32)
        for r in range(12, 16):
            for g in range(CHUNK // 16):
                fbuf[r, pl.ds(g * 16, 16)] = zv

        ii = lax.iota(i32, 16)

        @pl.loop(np.int32(0), np.int32(nchunk), step=np.int32(1))
        def _chunk(j):
            jj = j.astype(i32)
            off = base + jj * np.int32(CHUNK)
            for g in range(CHUNK // 16):
                eidx_v[pl.ds(g * 16, 16)] = ii + (off + np.int32(g * 16))
            c1 = pltpu.async_copy(psr.at[eidx_v], psr_v, sem)
            c2 = pltpu.async_copy(ridx.at[eidx_v], ridx_v, sem)
            c1.wait(); c2.wait()
            for g in range(CHUNK // 16):
                sl = pl.ds(g * 16, 16)
                p = psr_v[sl]
                sub = lax.shift_right_logical(p, 9)
                rel = lax.bitwise_and(p, np.int32(511))
                s10 = sub * np.int32(10)
                r10 = rel * np.int32(10)
                q8 = ridx_v[sl] * np.int32(8)
                for k in range(8):
                    v = (plsc.load_gather(nrec_t, [s10 + np.int32(k)])
                         + plsc.load_gather(rrec_t, [r10 + np.int32(k)])
                         + plsc.load_gather(qrec_t, [q8 + np.int32(k)]))
                    fbuf[k, sl] = v
                fbuf[8, sl] = plsc.load_gather(nrec_t, [s10 + np.int32(8)])
                fbuf[9, sl] = plsc.load_gather(nrec_t, [s10 + np.int32(9)])
                fbuf[10, sl] = plsc.load_gather(rrec_t, [r10 + np.int32(8)])
                fbuf[11, sl] = plsc.load_gather(rrec_t, [r10 + np.int32(9)])
            oidx = ii * np.int32(nrow) + (rbase + jj)
            pltpu.sync_copy(fbuf, o2.at[oidx])

    return gather


def _edge_body(wa_ref, x_ref, o_ref):
    x = x_ref[...]
    logit = jnp.zeros(x.shape[1:], f32)
    for k in range(8):
        logit = logit + wa_ref[0, k] * jax.nn.relu(x[k])
    alpha = 1.0 / (1.0 + jnp.exp(-logit))
    fs, ns2, gr, nr2 = x[8], x[9], x[10], x[11]
    x2 = fs * fs * ns2
    y2 = gr * gr * nr2
    t = 1.0 + C * y2
    den = jnp.maximum(1.0 + C * C * x2 * y2, MIN_NORM)
    aa = t / den * fs
    bb = (1.0 - C * x2) / den * gr
    m2 = aa * aa * ns2 + bb * bb * nr2
    pn = jnp.maximum(jnp.sqrt(m2), MIN_NORM)
    pf = jnp.where(pn > MAXNORM, MAXNORM / pn, 1.0)
    scale = _logmap0_scale(pf * pf * m2) * pf * alpha
    o_ref[0] = scale * aa
    o_ref[1] = scale * bb


def _edge_call(wa, o3, e_pad):
    nrow = e_pad // CHUNK
    br = 256
    nblk = nrow // br
    return pl.pallas_call(
        _edge_body,
        grid=(nblk,),
        in_specs=[
            pl.BlockSpec(memory_space=pltpu.SMEM),
            pl.BlockSpec((16, br, CHUNK), lambda i: (0, i, 0)),
        ],
        out_specs=pl.BlockSpec((2, br, CHUNK), lambda i: (0, i, 0)),
        out_shape=jax.ShapeDtypeStruct((2, nrow, CHUNK), f32),
    )(wa, o3)


def _make_scatter(e_pad):
    per_tile = e_pad // NW
    nchunk = per_tile // CHUNK
    rows_per_tile = A_PAD // NS
    mesh = plsc.VectorSubcoreMesh(core_axis_name="c", subcore_axis_name="s")

    @functools.partial(
        pl.kernel,
        out_type=jax.ShapeDtypeStruct((NC, NS, rows_per_tile, IN_DIM), f32),
        mesh=mesh,
        compiler_params=_SC_PARAMS,
        scratch_types=[
            pltpu.VMEM((CHUNK,), i32),
            pltpu.VMEM((CHUNK,), i32),
            pltpu.VMEM((CHUNK,), i32),
            pltpu.VMEM((CHUNK,), i32),
            pltpu.VMEM((CHUNK,), i32),
            pltpu.VMEM((CHUNK,), f32),
            pltpu.VMEM((CHUNK,), f32),
            pltpu.VMEM((CHUNK, IN_DIM), f32),
            pltpu.VMEM((CHUNK, IN_DIM), f32),
            pltpu.VMEM((CHUNK, IN_DIM), f32),
            pltpu.VMEM_SHARED((A_PAD, IN_DIM), f32),
            pltpu.SemaphoreType.DMA,
        ],
    )
    def scatter(hid, rela, psr, obj, w1f, w2f, out,
                eidx_v, psr_v, obj_v, sub_v, rel_v, w1_v, w2_v,
                hs_b, hr_b, msg_b, acc, sem):
        sid = lax.axis_index("s").astype(i32)
        cid = lax.axis_index("c").astype(i32)
        wid = sid * np.int32(NC) + cid
        base = wid * np.int32(per_tile)
        row0 = sid * np.int32(rows_per_tile)
        ii = lax.iota(i32, 16)

        zv = jnp.zeros((16,), f32)
        for e in range(CHUNK):
            for g in range(IN_DIM // 16):
                msg_b[e, pl.ds(g * 16, 16)] = zv

        @pl.loop(np.int32(0), np.int32(4), step=np.int32(1))
        def _zero(k):
            r = row0 + k.astype(i32) * np.int32(CHUNK)
            pltpu.sync_copy(msg_b, acc.at[pl.ds(r, CHUNK)])
        pltpu.sync_copy(msg_b.at[pl.ds(0, rows_per_tile - 4 * CHUNK)],
                        acc.at[pl.ds(row0 + np.int32(4 * CHUNK),
                                     rows_per_tile - 4 * CHUNK)])

        plsc.subcore_barrier()

        @pl.loop(np.int32(0), np.int32(nchunk), step=np.int32(1))
        def _chunk(j):
            off = base + j.astype(i32) * np.int32(CHUNK)
            for g in range(CHUNK // 16):
                eidx_v[pl.ds(g * 16, 16)] = ii + (off + np.int32(g * 16))
            c1 = pltpu.async_copy(psr.at[eidx_v], psr_v, sem)
            c2 = pltpu.async_copy(obj.at[eidx_v], obj_v, sem)
            c3 = pltpu.async_copy(w1f.at[eidx_v], w1_v, sem)
            c4 = pltpu.async_copy(w2f.at[eidx_v], w2_v, sem)
            c1.wait(); c2.wait(); c3.wait(); c4.wait()
            for g in range(CHUNK // 16):
                sl = pl.ds(g * 16, 16)
                p = psr_v[sl]
                sub_v[sl] = lax.shift_right_logical(p, 9)
                rel_v[sl] = lax.bitwise_and(p, np.int32(511))
            q = CHUNK // 8
            hs_g = []
            for piece in range(8):
                o = piece * q
                hs_g.append(pltpu.async_copy(
                    hid.at[sub_v.at[pl.ds(o, q)]], hs_b.at[pl.ds(o, q)], sem))
                hs_g.append(pltpu.async_copy(
                    rela.at[rel_v.at[pl.ds(o, q)]], hr_b.at[pl.ds(o, q)], sem))

            for piece in range(8):
                hs_g[2 * piece].wait()
                hs_g[2 * piece + 1].wait()

                @pl.loop(np.int32(piece * q), np.int32((piece + 1) * q),
                         step=np.int32(1))
                def _edge(e):
                    ee = jnp.full((16,), e, dtype=i32)
                    w1v = plsc.load_gather(w1_v, [ee])
                    w2v = plsc.load_gather(w2_v, [ee])
                    for k in range(IN_DIM // 16):
                        sl = pl.ds(k * 16, 16)
                        msg_b[e, sl] = w1v * hs_b[e, sl] + w2v * hr_b[e, sl]

            pltpu.sync_copy(msg_b, acc.at[obj_v], add=True)

        plsc.subcore_barrier()
        pltpu.sync_copy(acc.at[pl.ds(row0, rows_per_tile)], out.at[cid].at[sid])

    return scatter


def _final_body(p_ref, wh_ref, o_ref):
    g = p_ref[0] + p_ref[1]
    a = lax.dot_general(g, wh_ref[...], (((1,), (1,)), ((), ())),
                        preferred_element_type=f32)
    an2 = jnp.sum(a * a, axis=-1, keepdims=True)
    fa = _expmap0_scale(an2)
    y = fa * a
    o_ref[...] = y * _logmap0_scale(fa * fa * an2)


def _final_call(parts, w_h):
    blk = 632
    nblk = 10112 // blk
    return pl.pallas_call(
        _final_body,
        grid=(nblk,),
        in_specs=[
            pl.BlockSpec((NC, blk, IN_DIM), lambda i: (0, i, 0)),
            pl.BlockSpec((IN_DIM, IN_DIM), lambda i: (0, 0)),
        ],
        out_specs=pl.BlockSpec((blk, IN_DIM), lambda i: (i, 0)),
        out_shape=jax.ShapeDtypeStruct((10112, IN_DIM), f32),
    )(parts, w_h)


def kernel(q_sub, q_rel, hidden, edges, n_node, old_nodes_new_idx, rela_embed,
           Ws_attn_W, Wr_attn_W, Wqr_attn_W, Wqr_attn_b, W_attn_W, W_h_W):
    with jax.enable_x64(False):
        return _run(q_sub, q_rel, hidden, edges, n_node, old_nodes_new_idx,
                    rela_embed, Ws_attn_W, Wr_attn_W, Wqr_attn_W, Wqr_attn_b,
                    W_attn_W, W_h_W)


def _run(q_sub, q_rel, hidden, edges, n_node, old_nodes_new_idx, rela_embed,
         Ws_attn_W, Wr_attn_W, Wqr_attn_W, Wqr_attn_b, W_attn_W, W_h_W):
    hidden = hidden.astype(f32)
    rela = rela_embed.astype(f32)
    n_hid = hidden.shape[0]
    n_rel = rela.shape[0]
    n_edge = edges.shape[0]
    e_pad = ((n_edge + NW * CHUNK * 8 - 1) // (NW * CHUNK * 8)) * (NW * CHUNK * 8)

    h_pad = jnp.pad(hidden, ((0, N_PAD - n_hid), (0, 0)))
    rela_pad = jnp.pad(rela, ((0, R_PAD - n_rel), (0, 0)))
    ws_pad = jnp.pad(Ws_attn_W.astype(f32).T, ((0, 0), (0, 8)))
    wr_pad = jnp.pad(Wr_attn_W.astype(f32).T, ((0, 0), (0, 8)))
    wqr_pad = jnp.pad(Wqr_attn_W.astype(f32).T, ((0, 0), (0, 8)))
    b_pad = jnp.pad(Wqr_attn_b.astype(f32), (0, 8)).reshape(1, 16)
    wa = W_attn_W.astype(f32)
    q_rel32 = q_rel.astype(i32).reshape(512, 1)

    npad = e_pad - n_edge
    sub_c = edges[:, 4].astype(i32)
    rel_c = edges[:, 2].astype(i32)
    psr = jnp.concatenate([sub_c * R_PAD + rel_c, jnp.zeros((npad,), i32)])
    ridx = jnp.concatenate([edges[:, 0].astype(i32), jnp.zeros((npad,), i32)])
    obj = jnp.concatenate([
        (edges[:, 5] + (n_node - n_hid)).astype(i32),
        jnp.full((npad,), N_NODE, i32),
    ])

    nrec = _prep_call(h_pad, ws_pad)
    rrec, qrec = _relprep_call(rela_pad, wr_pad, wqr_pad, b_pad, q_rel32)

    o2 = _make_gather(e_pad)(
        nrec.reshape(-1), rrec.reshape(-1), qrec.reshape(-1), psr, ridx)

    nrow = e_pad // CHUNK
    w2d = _edge_call(wa, o2.reshape(16, nrow, CHUNK), e_pad)

    parts = _make_scatter(e_pad)(
        hidden, rela, psr, obj, w2d[0].reshape(-1), w2d[1].reshape(-1))

    parts = jnp.pad(parts.reshape(NC, A_PAD, IN_DIM),
                    ((0, 0), (0, 10112 - A_PAD), (0, 0)))
    out = _final_call(parts, W_h_W.astype(f32))
    return out[:n_hid]


# speedup vs baseline: 1.3644x; 1.0215x over previous
import functools

import jax
import jax.numpy as jnp
import numpy as np
from jax import lax
from jax.experimental import pallas as pl
from jax.experimental.pallas import tpu as pltpu
from jax.experimental.pallas import tpu_sc as plsc

f32 = jnp.float32
i32 = jnp.int32

MIN_NORM = 1e-15
C = float(np.float32(1e-6))
SQRT_C = float(np.sqrt(np.float32(1e-6)))
MAXNORM = float(np.float32(1.0 - 0.004) / np.float32(SQRT_C))

N_NODE = 10000
N_PAD = 10240
A_PAD = 10112
R_PAD = 512
IN_DIM = 128
NC, NS = 2, 16
NW = NC * NS
CHUNK = 128

_SC_PARAMS = pltpu.CompilerParams(needs_layout_passes=False)


def _expmap0_scale(ns2):
    ns = jnp.sqrt(ns2)
    un = jnp.maximum(ns, MIN_NORM)
    arg = jnp.clip(SQRT_C * un, -15.0, 15.0)
    g = jnp.tanh(arg) / (SQRT_C * un)
    gn = jnp.maximum(g * ns, MIN_NORM)
    return g * jnp.where(gn > MAXNORM, MAXNORM / gn, 1.0)


def _logmap0_scale(yn2):
    yn = jnp.maximum(jnp.sqrt(yn2), MIN_NORM)
    z = jnp.clip(SQRT_C * yn, -1 + 1e-05, 1 - 1e-05)
    art = 0.5 * jnp.log((1.0 + z) / (1.0 - z))
    return art / (yn * SQRT_C)


def _prep_body(h_ref, ws_ref, rec_ref):
    h = h_ref[...]
    a = jnp.dot(h, ws_ref[...], preferred_element_type=f32)
    ns2 = jnp.sum(h * h, axis=-1, keepdims=True)
    fs = _expmap0_scale(ns2)
    lane = lax.broadcasted_iota(i32, (1, 16), 1)
    rec = a + jnp.where(lane == 8, fs, 0.0) + jnp.where(lane == 9, ns2, 0.0)
    rec_ref[...] = lax.slice(rec, (0, 0), (rec.shape[0], 10))


def _prep_call(h_pad, ws_pad):
    nblk = N_PAD // 512
    return pl.pallas_call(
        _prep_body,
        grid=(nblk,),
        in_specs=[
            pl.BlockSpec((512, IN_DIM), lambda i: (i, 0)),
            pl.BlockSpec((IN_DIM, 16), lambda i: (0, 0)),
        ],
        out_specs=pl.BlockSpec((512, 10), lambda i: (i, 0)),
        out_shape=jax.ShapeDtypeStruct((N_PAD, 10), f32),
    )(h_pad, ws_pad)


def _relprep_body(rp_ref, wr_ref, wqr_ref, b_ref, q_ref, rrec_ref, qrec_ref):
    rp = rp_ref[...]
    a = jnp.dot(rp, wr_ref[...], preferred_element_type=f32)
    nr2 = jnp.sum(rp * rp, axis=-1, keepdims=True)
    gr = _expmap0_scale(nr2)
    lane = lax.broadcasted_iota(i32, (1, 16), 1)
    rec = a + jnp.where(lane == 8, gr, 0.0) + jnp.where(lane == 9, nr2, 0.0)
    rrec_ref[...] = lax.slice(rec, (0, 0), (512, 10))
    col = lax.broadcasted_iota(i32, (512, R_PAD), 1)
    oh = jnp.where(col == q_ref[...], 1.0, 0.0).astype(f32)
    hq = jnp.dot(oh, rp, preferred_element_type=f32)
    qrec = jnp.dot(hq, wqr_ref[...], preferred_element_type=f32) + b_ref[...]
    qrec_ref[...] = lax.slice(qrec, (0, 0), (512, 8))


def _relprep_call(rela_pad, wr_pad, wqr_pad, b_pad, q_rel32):
    return pl.pallas_call(
        _relprep_body,
        in_specs=[
            pl.BlockSpec((R_PAD, IN_DIM), lambda: (0, 0)),
            pl.BlockSpec((IN_DIM, 16), lambda: (0, 0)),
            pl.BlockSpec((IN_DIM, 16), lambda: (0, 0)),
            pl.BlockSpec((1, 16), lambda: (0, 0)),
            pl.BlockSpec((512, 1), lambda: (0, 0)),
        ],
        out_specs=[
            pl.BlockSpec((R_PAD, 10), lambda: (0, 0)),
            pl.BlockSpec((512, 8), lambda: (0, 0)),
        ],
        out_shape=[
            jax.ShapeDtypeStruct((R_PAD, 10), f32),
            jax.ShapeDtypeStruct((512, 8), f32),
        ],
    )(rela_pad, wr_pad, wqr_pad, b_pad, q_rel32)


def _make_gather(e_pad):
    per_tile = e_pad // NW
    nchunk = per_tile // CHUNK
    nrow = e_pad // CHUNK
    mesh = plsc.VectorSubcoreMesh(core_axis_name="c", subcore_axis_name="s")

    @functools.partial(
        pl.kernel,
        out_type=jax.ShapeDtypeStruct((16 * nrow, CHUNK), f32),
        mesh=mesh,
        compiler_params=_SC_PARAMS,
        scratch_types=[
            pltpu.VMEM((N_PAD * 10,), f32),
            pltpu.VMEM((R_PAD * 10,), f32),
            pltpu.VMEM((512 * 8,), f32),
            pltpu.VMEM((CHUNK,), i32),
            pltpu.VMEM((CHUNK,), i32),
            pltpu.VMEM((CHUNK,), i32),
            pltpu.VMEM((16, CHUNK), f32),
            pltpu.SemaphoreType.DMA,
        ],
    )
    def gather(nflat, rflat, qflat, psr, ridx, o2,
               nrec_t, rrec_t, qrec_t, eidx_v, psr_v, ridx_v, fbuf, sem):
        sid = lax.axis_index("s").astype(i32)
        cid = lax.axis_index("c").astype(i32)
        wid = sid * np.int32(NC) + cid
        base = wid * np.int32(per_tile)
        rbase = wid * np.int32(nchunk)

        pltpu.sync_copy(nflat, nrec_t)
        pltpu.sync_copy(rflat, rrec_t)
        pltpu.sync_copy(qflat, qrec_t)

        zv = jnp.zeros((16,), f32)
        for r in range(12, 16):
            for g in range(CHUNK // 16):
                fbuf[r, pl.ds(g * 16, 16)] = zv

        ii = lax.iota(i32, 16)

        @pl.loop(np.int32(0), np.int32(nchunk), step=np.int32(1))
        def _chunk(j):
            jj = j.astype(i32)
            off = base + jj * np.int32(CHUNK)
            for g in range(CHUNK // 16):
                eidx_v[pl.ds(g * 16, 16)] = ii + (off + np.int32(g * 16))
            c1 = pltpu.async_copy(psr.at[eidx_v], psr_v, sem)
            c2 = pltpu.async_copy(ridx.at[eidx_v], ridx_v, sem)
            c1.wait(); c2.wait()
            for g in range(CHUNK // 16):
                sl = pl.ds(g * 16, 16)
                p = psr_v[sl]
                sub = lax.shift_right_logical(p, 9)
                rel = lax.bitwise_and(p, np.int32(511))
                s10 = sub * np.int32(10)
                r10 = rel * np.int32(10)
                q8 = ridx_v[sl] * np.int32(8)
                for k in range(8):
                    v = (plsc.load_gather(nrec_t, [s10 + np.int32(k)])
                         + plsc.load_gather(rrec_t, [r10 + np.int32(k)])
                         + plsc.load_gather(qrec_t, [q8 + np.int32(k)]))
                    fbuf[k, sl] = v
                fbuf[8, sl] = plsc.load_gather(nrec_t, [s10 + np.int32(8)])
                fbuf[9, sl] = plsc.load_gather(nrec_t, [s10 + np.int32(9)])
                fbuf[10, sl] = plsc.load_gather(rrec_t, [r10 + np.int32(8)])
                fbuf[11, sl] = plsc.load_gather(rrec_t, [r10 + np.int32(9)])
            oidx = ii * np.int32(nrow) + (rbase + jj)
            pltpu.sync_copy(fbuf, o2.at[oidx])

    return gather


def _edge_body(wa_ref, x_ref, o_ref):
    x = x_ref[...]
    logit = jnp.zeros(x.shape[1:], f32)
    for k in range(8):
        logit = logit + wa_ref[0, k] * jax.nn.relu(x[k])
    alpha = 1.0 / (1.0 + jnp.exp(-logit))
    fs, ns2, gr, nr2 = x[8], x[9], x[10], x[11]
    x2 = fs * fs * ns2
    y2 = gr * gr * nr2
    t = 1.0 + C * y2
    den = jnp.maximum(1.0 + C * C * x2 * y2, MIN_NORM)
    aa = t / den * fs
    bb = (1.0 - C * x2) / den * gr
    m2 = aa * aa * ns2 + bb * bb * nr2
    pn = jnp.maximum(jnp.sqrt(m2), MIN_NORM)
    pf = jnp.where(pn > MAXNORM, MAXNORM / pn, 1.0)
    scale = _logmap0_scale(pf * pf * m2) * pf * alpha
    o_ref[0] = scale * aa
    o_ref[1] = scale * bb


def _edge_call(wa, o3, e_pad):
    nrow = e_pad // CHUNK
    br = 256
    nblk = nrow // br
    return pl.pallas_call(
        _edge_body,
        grid=(nblk,),
        in_specs=[
            pl.BlockSpec(memory_space=pltpu.SMEM),
            pl.BlockSpec((16, br, CHUNK), lambda i: (0, i, 0)),
        ],
        out_specs=pl.BlockSpec((2, br, CHUNK), lambda i: (0, i, 0)),
        out_shape=jax.ShapeDtypeStruct((2, nrow, CHUNK), f32),
    )(wa, o3)


def _make_scatter(e_pad):
    per_tile = e_pad // NW
    nchunk = per_tile // CHUNK
    rows_per_tile = A_PAD // NS
    mesh = plsc.VectorSubcoreMesh(core_axis_name="c", subcore_axis_name="s")

    @functools.partial(
        pl.kernel,
        out_type=jax.ShapeDtypeStruct((NC, NS, rows_per_tile, IN_DIM), f32),
        mesh=mesh,
        compiler_params=_SC_PARAMS,
        scratch_types=[
            pltpu.VMEM((CHUNK,), i32),
            pltpu.VMEM((CHUNK,), i32),
            pltpu.VMEM((CHUNK,), i32),
            pltpu.VMEM((CHUNK,), i32),
            pltpu.VMEM((CHUNK,), i32),
            pltpu.VMEM((CHUNK,), f32),
            pltpu.VMEM((CHUNK,), f32),
            pltpu.VMEM((CHUNK, IN_DIM), f32),
            pltpu.VMEM((CHUNK, IN_DIM), f32),
            pltpu.VMEM((CHUNK, IN_DIM), f32),
            pltpu.VMEM_SHARED((A_PAD, IN_DIM), f32),
            pltpu.SemaphoreType.DMA,
        ],
    )
    def scatter(hid, rela, psr, obj, w1f, w2f, out,
                eidx_v, psr_v, obj_v, sub_v, rel_v, w1_v, w2_v,
                hs_b, hr_b, msg_b, acc, sem):
        sid = lax.axis_index("s").astype(i32)
        cid = lax.axis_index("c").astype(i32)
        wid = sid * np.int32(NC) + cid
        base = wid * np.int32(per_tile)
        row0 = sid * np.int32(rows_per_tile)
        ii = lax.iota(i32, 16)

        zv = jnp.zeros((16,), f32)
        for e in range(CHUNK):
            for g in range(IN_DIM // 16):
                msg_b[e, pl.ds(g * 16, 16)] = zv

        @pl.loop(np.int32(0), np.int32(4), step=np.int32(1))
        def _zero(k):
            r = row0 + k.astype(i32) * np.int32(CHUNK)
            pltpu.sync_copy(msg_b, acc.at[pl.ds(r, CHUNK)])
        pltpu.sync_copy(msg_b.at[pl.ds(0, rows_per_tile - 4 * CHUNK)],
                        acc.at[pl.ds(row0 + np.int32(4 * CHUNK),
                                     rows_per_tile - 4 * CHUNK)])

        plsc.subcore_barrier()

        @pl.loop(np.int32(0), np.int32(nchunk), step=np.int32(1))
        def _chunk(j):
            off = base + j.astype(i32) * np.int32(CHUNK)
            for g in range(CHUNK // 16):
                eidx_v[pl.ds(g * 16, 16)] = ii + (off + np.int32(g * 16))
            c1 = pltpu.async_copy(psr.at[eidx_v], psr_v, sem)
            c2 = pltpu.async_copy(obj.at[eidx_v], obj_v, sem)
            c3 = pltpu.async_copy(w1f.at[eidx_v], w1_v, sem)
            c4 = pltpu.async_copy(w2f.at[eidx_v], w2_v, sem)
            c1.wait(); c2.wait(); c3.wait(); c4.wait()
            for g in range(CHUNK // 16):
                sl = pl.ds(g * 16, 16)
                p = psr_v[sl]
                sub_v[sl] = lax.shift_right_logical(p, 9)
                rel_v[sl] = lax.bitwise_and(p, np.int32(511))
            q = CHUNK // 16
            hs_g = []
            for piece in range(16):
                o = piece * q
                hs_g.append(pltpu.async_copy(
                    hid.at[sub_v.at[pl.ds(o, q)]], hs_b.at[pl.ds(o, q)], sem))
                hs_g.append(pltpu.async_copy(
                    rela.at[rel_v.at[pl.ds(o, q)]], hr_b.at[pl.ds(o, q)], sem))

            for piece in range(16):
                hs_g[2 * piece].wait()
                hs_g[2 * piece + 1].wait()

                @pl.loop(np.int32(piece * q), np.int32((piece + 1) * q),
                         step=np.int32(1))
                def _edge(e):
                    ee = jnp.full((16,), e, dtype=i32)
                    w1v = plsc.load_gather(w1_v, [ee])
                    w2v = plsc.load_gather(w2_v, [ee])
                    for k in range(IN_DIM // 16):
                        sl = pl.ds(k * 16, 16)
                        msg_b[e, sl] = w1v * hs_b[e, sl] + w2v * hr_b[e, sl]

            pltpu.sync_copy(msg_b, acc.at[obj_v], add=True)

        plsc.subcore_barrier()
        pltpu.sync_copy(acc.at[pl.ds(row0, rows_per_tile)], out.at[cid].at[sid])

    return scatter


def _final_body(p_ref, wh_ref, o_ref):
    g = p_ref[0] + p_ref[1]
    a = lax.dot_general(g, wh_ref[...], (((1,), (1,)), ((), ())),
                        preferred_element_type=f32)
    an2 = jnp.sum(a * a, axis=-1, keepdims=True)
    fa = _expmap0_scale(an2)
    y = fa * a
    o_ref[...] = y * _logmap0_scale(fa * fa * an2)


def _final_call(parts, w_h):
    blk = 632
    nblk = 10112 // blk
    return pl.pallas_call(
        _final_body,
        grid=(nblk,),
        in_specs=[
            pl.BlockSpec((NC, blk, IN_DIM), lambda i: (0, i, 0)),
            pl.BlockSpec((IN_DIM, IN_DIM), lambda i: (0, 0)),
        ],
        out_specs=pl.BlockSpec((blk, IN_DIM), lambda i: (i, 0)),
        out_shape=jax.ShapeDtypeStruct((10112, IN_DIM), f32),
    )(parts, w_h)


def kernel(q_sub, q_rel, hidden, edges, n_node, old_nodes_new_idx, rela_embed,
           Ws_attn_W, Wr_attn_W, Wqr_attn_W, Wqr_attn_b, W_attn_W, W_h_W):
    with jax.enable_x64(False):
        return _run(q_sub, q_rel, hidden, edges, n_node, old_nodes_new_idx,
                    rela_embed, Ws_attn_W, Wr_attn_W, Wqr_attn_W, Wqr_attn_b,
                    W_attn_W, W_h_W)


def _run(q_sub, q_rel, hidden, edges, n_node, old_nodes_new_idx, rela_embed,
         Ws_attn_W, Wr_attn_W, Wqr_attn_W, Wqr_attn_b, W_attn_W, W_h_W):
    hidden = hidden.astype(f32)
    rela = rela_embed.astype(f32)
    n_hid = hidden.shape[0]
    n_rel = rela.shape[0]
    n_edge = edges.shape[0]
    e_pad = ((n_edge + NW * CHUNK * 8 - 1) // (NW * CHUNK * 8)) * (NW * CHUNK * 8)

    h_pad = jnp.pad(hidden, ((0, N_PAD - n_hid), (0, 0)))
    rela_pad = jnp.pad(rela, ((0, R_PAD - n_rel), (0, 0)))
    ws_pad = jnp.pad(Ws_attn_W.astype(f32).T, ((0, 0), (0, 8)))
    wr_pad = jnp.pad(Wr_attn_W.astype(f32).T, ((0, 0), (0, 8)))
    wqr_pad = jnp.pad(Wqr_attn_W.astype(f32).T, ((0, 0), (0, 8)))
    b_pad = jnp.pad(Wqr_attn_b.astype(f32), (0, 8)).reshape(1, 16)
    wa = W_attn_W.astype(f32)
    q_rel32 = q_rel.astype(i32).reshape(512, 1)

    npad = e_pad - n_edge
    sub_c = edges[:, 4].astype(i32)
    rel_c = edges[:, 2].astype(i32)
    psr = jnp.concatenate([sub_c * R_PAD + rel_c, jnp.zeros((npad,), i32)])
    ridx = jnp.concatenate([edges[:, 0].astype(i32), jnp.zeros((npad,), i32)])
    obj = jnp.concatenate([
        (edges[:, 5] + (n_node - n_hid)).astype(i32),
        jnp.full((npad,), N_NODE, i32),
    ])

    nrec = _prep_call(h_pad, ws_pad)
    rrec, qrec = _relprep_call(rela_pad, wr_pad, wqr_pad, b_pad, q_rel32)

    o2 = _make_gather(e_pad)(
        nrec.reshape(-1), rrec.reshape(-1), qrec.reshape(-1), psr, ridx)

    nrow = e_pad // CHUNK
    w2d = _edge_call(wa, o2.reshape(16, nrow, CHUNK), e_pad)

    parts = _make_scatter(e_pad)(
        hidden, rela, psr, obj, w2d[0].reshape(-1), w2d[1].reshape(-1))

    parts = jnp.pad(parts.reshape(NC, A_PAD, IN_DIM),
                    ((0, 0), (0, 10112 - A_PAD), (0, 0)))
    out = _final_call(parts, W_h_W.astype(f32))
    return out[:n_hid]
